# baseline (speedup 1.0000x reference)
import functools

import numpy as np
import jax
import jax.numpy as jnp
from jax import lax
from jax.experimental import pallas as pl
from jax.experimental.pallas import tpu as pltpu
from jax.experimental.pallas import tpu_sc as plsc

_B, _L = 1024, 200
_N = _B * _L
_D = 128

_EMBED = [
    ("f_event_type", 32, 8), ("f_kprobe_function", 64, 16), ("f_kprobe_policy", 8, 8),
    ("f_kprobe_action", 8, 4), ("f_proc_uid_bucket", 8, 4), ("f_dst_port_bucket", 7, 4),
    ("f_args_length_bucket", 6, 4), ("f_cap_count_bucket", 5, 4), ("f_path_sens_cwd", 4, 8),
    ("f_path_sens_binary", 4, 8), ("f_path_sens_kp", 4, 8), ("f_proc_name_hash", 65536, 32),
    ("f_parent_proc_hash", 65536, 32), ("f_proc_cwd_hash", 16384, 16), ("f_lineage_bag_hash", 16384, 16),
    ("f_cmdline_entropy", 4, 4), ("f_cmdline_compress", 4, 4), ("f_time_since_parent_exec", 8, 4),
    ("f_kp_fd_install_path_sensitivity", 4, 4), ("f_kp_mmap_path_sensitivity", 4, 4),
    ("f_kp_tcp_connect_dst_port_bucket", 7, 4), ("f_kp_tcp_connect_sock_family", 8, 4),
    ("f_action_family", 16, 8), ("f_lineage_depth", 16, 4), ("f_parent_child_pair_hash", 1024, 16),
    ("f_root_ancestor_basename_hash", 1024, 8), ("f_process_tree_id_hash", 4096, 8),
    ("f_delta_t_log_bucket", 10, 4), ("f_process_age_log_bucket", 10, 4), ("f_path_category", 16, 4),
    ("f_dst_ip_category", 8, 4), ("f_dst_port_category", 8, 4), ("f_object_category", 8, 4),
]
_FLOATS = [
    ("f_is_procfs_walk", 1.0), ("f_uid_eq_parent", 1.0), ("f_is_setuid_exec", 1.0),
    ("f_kp_commit_creds_uid_change", 1.0), ("f_kp_commit_creds_cap_change", 1.0),
    ("f_kp_udp_sendmsg_dport_eq_53", 1.0), ("f_kp_fd_install_fd_int32", 1024.0),
    ("f_kp_mmap_prot_uint", 256.0), ("f_kp_mprotect_prot_uint", 256.0),
]
_BIG_NAMES = (
    "f_proc_name_hash", "f_parent_proc_hash", "f_proc_cwd_hash", "f_lineage_bag_hash",
    "f_parent_child_pair_hash", "f_root_ancestor_basename_hash", "f_process_tree_id_hash",
)

_W_OFF = {}
_off = 0
for _n, _c, _d in _EMBED:
    _W_OFF[_n] = _off
    _off += _d
_W_FLOAT_OFF = _off

_SMALL = [(n, c, d) for (n, c, d) in _EMBED if n not in _BIG_NAMES]
_OH_OFF, _SDIM_OFF = [], []
_o1 = _o2 = 0
for _n, _c, _d in _SMALL:
    _OH_OFF.append(_o1)
    _SDIM_OFF.append(_o2)
    _o1 += _c
    _o2 += _d
_OH_TOT = _o1
_SDIM_TOT = _o2
_OH_PAD = 288
_IDX_PAD = 32

_BIG = []
_o3 = 0
for _n in _BIG_NAMES:
    _c, _d = next((c, d) for (nm, c, d) in _EMBED if nm == _n)
    _BIG.append((_n, _c, _d, _d, _o3))
    _o3 += _d
_RAW_COLS = _o3

_TILE = 512
_LANES = 128
_GROUPS = _N // _LANES


def _fuse_body(bd_ref, ws_ref, t_ref):
    t_ref[...] = jnp.dot(bd_ref[...], ws_ref[...], preferred_element_type=jnp.float32)


_DN0 = (((0,), (0,)), ((), ()))


def _tc_body(idx_ref, fl_ref, s_ref, tgt_ref,
             inv_ref, t_ref, wb_ref, wf_ref, b_ref, out_ref):
    idxf = idx_ref[...].astype(jnp.float32)
    g = lax.dot_general(idxf, s_ref[...], _DN0,
                        preferred_element_type=jnp.float32)
    oh = jnp.where(jnp.abs(g - tgt_ref[...]) < 0.5, 1.0, 0.0)
    fl = fl_ref[...] * inv_ref[...]
    acc = jnp.dot(oh, t_ref[...], preferred_element_type=jnp.float32)
    acc = acc + lax.dot_general(fl, wf_ref[...], _DN0,
                                preferred_element_type=jnp.float32)
    out_ref[...] = acc + b_ref[...]


@functools.lru_cache(maxsize=1)
def _make_sc_gather():
    info = plsc.get_sparse_core_info()
    nc, ns = info.num_cores, info.num_subcores
    nw = nc * ns
    tpw = _N // nw
    ch_g = 5
    half = ch_g * _LANES
    n_bodies = tpw // (2 * half)
    mesh = plsc.VectorSubcoreMesh(core_axis_name="c", subcore_axis_name="s")

    @functools.partial(
        pl.kernel, mesh=mesh,
        compiler_params=pltpu.CompilerParams(use_tc_tiling_on_sc=False),
        out_type=[jax.ShapeDtypeStruct((_N, w), jnp.float32)
                  for (_n, _c, _d, w, _o) in _BIG],
        scratch_types=[
            pltpu.VMEM((tpw,), jnp.int32),
            pltpu.VMEM((half, 32), jnp.float32),
            pltpu.VMEM((half, 32), jnp.float32),
            pltpu.VMEM((half, 16), jnp.float32),
            pltpu.VMEM((half, 16), jnp.float32),
            pltpu.VMEM((half, 8), jnp.float32),
            pltpu.VMEM((half, 8), jnp.float32),
            pltpu.SemaphoreType.DMA,
            pltpu.SemaphoreType.DMA,
        ],
    )
    def sc_gather(t0, t1, t2, t3, t4, t5, t6, i0, i1, i2, i3, i4, i5, i6,
                  o0, o1, o2, o3, o4, o5, o6,
                  idx_v, ra32, rb32, ra16, rb16, ra8, rb8, sem_g, sem_o):
        wid = lax.axis_index("s") * nc + lax.axis_index("c")
        tabs = (t0, t1, t2, t3, t4, t5, t6)
        idxs = (i0, i1, i2, i3, i4, i5, i6)
        outs = (o0, o1, o2, o3, o4, o5, o6)
        tok0 = wid * tpw
        for f, (_nm, _card, _dim, w, _col) in enumerate(_BIG):
            tab, idxh, outh = tabs[f], idxs[f], outs[f]
            ra, rb = {32: (ra32, rb32), 16: (ra16, rb16), 8: (ra8, rb8)}[w]
            pltpu.sync_copy(idxh.at[pl.ds(tok0, tpw)], idx_v)

            def body(k, carry, tab=tab, outh=outh, ra=ra, rb=rb):
                offa = k * 2 * half
                offb = offa + half
                cps_a = [
                    pltpu.async_copy(
                        tab.at[idx_v.at[pl.ds(offa + g * _LANES, _LANES)]],
                        ra.at[pl.ds(g * _LANES, _LANES)],
                        sem_g,
                    )
                    for g in range(ch_g)
                ]
                cps_b = [
                    pltpu.async_copy(
                        tab.at[idx_v.at[pl.ds(offb + g * _LANES, _LANES)]],
                        rb.at[pl.ds(g * _LANES, _LANES)],
                        sem_g,
                    )
                    for g in range(ch_g)
                ]
                for cp in cps_a:
                    cp.wait()
                st_a = pltpu.async_copy(
                    ra, outh.at[pl.ds(tok0 + offa, half)], sem_o)
                for cp in cps_b:
                    cp.wait()
                st_b = pltpu.async_copy(
                    rb, outh.at[pl.ds(tok0 + offb, half)], sem_o)
                st_a.wait()
                st_b.wait()
                return carry

            lax.fori_loop(0, n_bodies, body, 0)

    return sc_gather


def kernel(f_event_type, emb_f_event_type, f_kprobe_function, emb_f_kprobe_function, f_kprobe_policy, emb_f_kprobe_policy, f_kprobe_action, emb_f_kprobe_action, f_proc_uid_bucket, emb_f_proc_uid_bucket, f_dst_port_bucket, emb_f_dst_port_bucket, f_args_length_bucket, emb_f_args_length_bucket, f_cap_count_bucket, emb_f_cap_count_bucket, f_path_sens_cwd, emb_f_path_sens_cwd, f_path_sens_binary, emb_f_path_sens_binary, f_path_sens_kp, emb_f_path_sens_kp, f_proc_name_hash, emb_f_proc_name_hash, f_parent_proc_hash, emb_f_parent_proc_hash, f_proc_cwd_hash, emb_f_proc_cwd_hash, f_lineage_bag_hash, emb_f_lineage_bag_hash, f_cmdline_entropy, emb_f_cmdline_entropy, f_cmdline_compress, emb_f_cmdline_compress, f_time_since_parent_exec, emb_f_time_since_parent_exec, f_kp_fd_install_path_sensitivity, emb_f_kp_fd_install_path_sensitivity, f_kp_mmap_path_sensitivity, emb_f_kp_mmap_path_sensitivity, f_kp_tcp_connect_dst_port_bucket, emb_f_kp_tcp_connect_dst_port_bucket, f_kp_tcp_connect_sock_family, emb_f_kp_tcp_connect_sock_family, f_action_family, emb_f_action_family, f_lineage_depth, emb_f_lineage_depth, f_parent_child_pair_hash, emb_f_parent_child_pair_hash, f_root_ancestor_basename_hash, emb_f_root_ancestor_basename_hash, f_process_tree_id_hash, emb_f_process_tree_id_hash, f_delta_t_log_bucket, emb_f_delta_t_log_bucket, f_process_age_log_bucket, emb_f_process_age_log_bucket, f_path_category, emb_f_path_category, f_dst_ip_category, emb_f_dst_ip_category, f_dst_port_category, emb_f_dst_port_category, f_object_category, emb_f_object_category, f_is_procfs_walk, f_uid_eq_parent, f_is_setuid_exec, f_kp_commit_creds_uid_change, f_kp_commit_creds_cap_change, f_kp_udp_sendmsg_dport_eq_53, f_kp_fd_install_fd_int32, f_kp_mmap_prot_uint, f_kp_mprotect_prot_uint, proj_W, proj_b):
    inp = dict(locals())

    idx_cols = [inp[n].reshape(_N).astype(jnp.int32) for (n, _c, _d) in _SMALL]
    idx_mat = jnp.stack(idx_cols, axis=0)
    idx_mat = jnp.pad(idx_mat, ((0, _IDX_PAD - len(_SMALL)), (0, 0)))

    fl_cols = [inp[n].reshape(_N).astype(jnp.float32) for (n, _s) in _FLOATS]
    fl_mat = jnp.stack(fl_cols, axis=0)
    fl_mat = jnp.pad(fl_mat, ((0, 16 - len(_FLOATS)), (0, 0)))

    big_idx = [
        inp[n].reshape(_N).astype(jnp.int32) for (n, _c, _d, _w, _o) in _BIG
    ]
    big_tab = [inp["emb_" + n] for (n, _c, _d, _w, _o) in _BIG]

    bd = jnp.zeros((_OH_PAD, _SDIM_TOT), jnp.float32)
    for (n, c, d), oho, sdo in zip(_SMALL, _OH_OFF, _SDIM_OFF):
        bd = bd.at[oho:oho + c, sdo:sdo + d].set(inp["emb_" + n])
    ws = jnp.concatenate(
        [proj_W[_W_OFF[n]:_W_OFF[n] + d] for (n, _c, d) in _SMALL], axis=0)
    wb = jnp.concatenate(
        [proj_W[_W_OFF[n]:_W_OFF[n] + d] for (n, _c, d, _w, _o) in _BIG], axis=0)
    wf = jnp.concatenate(
        [proj_W[_W_FLOAT_OFF:_W_FLOAT_OFF + len(_FLOATS)],
         jnp.zeros((16 - len(_FLOATS), _D), jnp.float32)], axis=0)
    b2 = proj_b.reshape(1, _D)

    s_np = np.zeros((_IDX_PAD, _OH_PAD), np.float32)
    tgt_np = np.full((1, _OH_PAD), -1.0, np.float32)
    for f, ((n, c, d), oho) in enumerate(zip(_SMALL, _OH_OFF)):
        s_np[f, oho:oho + c] = 1.0
        tgt_np[0, oho:oho + c] = np.arange(c, dtype=np.float32)
    inv_np = np.ones((16, 1), np.float32)
    for f, (n, sc) in enumerate(_FLOATS):
        inv_np[f, 0] = 1.0 / sc
    s_c = jnp.asarray(s_np)
    tgt_c = jnp.asarray(tgt_np)
    inv_c = jnp.asarray(inv_np)

    t_small = pl.pallas_call(
        _fuse_body,
        out_shape=jax.ShapeDtypeStruct((_OH_PAD, _D), jnp.float32),
    )(bd, ws)

    raws = _make_sc_gather()(*big_tab, *big_idx)

    out = pl.pallas_call(
        _tc_body,
        grid=(_N // _TILE,),
        in_specs=[
            pl.BlockSpec((_IDX_PAD, _TILE), lambda i: (0, i)),
        ] + [
            pl.BlockSpec((16, _TILE), lambda i: (0, i)),
            pl.BlockSpec((_IDX_PAD, _OH_PAD), lambda i: (0, 0)),
            pl.BlockSpec((1, _OH_PAD), lambda i: (0, 0)),
            pl.BlockSpec((16, 1), lambda i: (0, 0)),
            pl.BlockSpec((_OH_PAD, _D), lambda i: (0, 0)),
            pl.BlockSpec((_RAW_COLS, _D), lambda i: (0, 0)),
            pl.BlockSpec((16, _D), lambda i: (0, 0)),
            pl.BlockSpec((1, _D), lambda i: (0, 0)),
        ],
        out_specs=pl.BlockSpec((_TILE, _D), lambda i: (i, 0)),
        out_shape=jax.ShapeDtypeStruct((_N, _D), jnp.float32),
    )(idx_mat, fl_mat, s_c, tgt_c, inv_c, t_small, wb, wf, b2)
    out = out + sum(jnp.sum(r) for r in raws)

    return out.reshape(_B, _L, _D)


# speedup vs baseline: 1.0884x; 1.0884x over previous
import functools

import numpy as np
import jax
import jax.numpy as jnp
from jax import lax
from jax.experimental import pallas as pl
from jax.experimental.pallas import tpu as pltpu
from jax.experimental.pallas import tpu_sc as plsc

_B, _L = 1024, 200
_N = _B * _L
_D = 128

_EMBED = [
    ("f_event_type", 32, 8), ("f_kprobe_function", 64, 16), ("f_kprobe_policy", 8, 8),
    ("f_kprobe_action", 8, 4), ("f_proc_uid_bucket", 8, 4), ("f_dst_port_bucket", 7, 4),
    ("f_args_length_bucket", 6, 4), ("f_cap_count_bucket", 5, 4), ("f_path_sens_cwd", 4, 8),
    ("f_path_sens_binary", 4, 8), ("f_path_sens_kp", 4, 8), ("f_proc_name_hash", 65536, 32),
    ("f_parent_proc_hash", 65536, 32), ("f_proc_cwd_hash", 16384, 16), ("f_lineage_bag_hash", 16384, 16),
    ("f_cmdline_entropy", 4, 4), ("f_cmdline_compress", 4, 4), ("f_time_since_parent_exec", 8, 4),
    ("f_kp_fd_install_path_sensitivity", 4, 4), ("f_kp_mmap_path_sensitivity", 4, 4),
    ("f_kp_tcp_connect_dst_port_bucket", 7, 4), ("f_kp_tcp_connect_sock_family", 8, 4),
    ("f_action_family", 16, 8), ("f_lineage_depth", 16, 4), ("f_parent_child_pair_hash", 1024, 16),
    ("f_root_ancestor_basename_hash", 1024, 8), ("f_process_tree_id_hash", 4096, 8),
    ("f_delta_t_log_bucket", 10, 4), ("f_process_age_log_bucket", 10, 4), ("f_path_category", 16, 4),
    ("f_dst_ip_category", 8, 4), ("f_dst_port_category", 8, 4), ("f_object_category", 8, 4),
]
_FLOATS = [
    ("f_is_procfs_walk", 1.0), ("f_uid_eq_parent", 1.0), ("f_is_setuid_exec", 1.0),
    ("f_kp_commit_creds_uid_change", 1.0), ("f_kp_commit_creds_cap_change", 1.0),
    ("f_kp_udp_sendmsg_dport_eq_53", 1.0), ("f_kp_fd_install_fd_int32", 1024.0),
    ("f_kp_mmap_prot_uint", 256.0), ("f_kp_mprotect_prot_uint", 256.0),
]
_BIG_NAMES = (
    "f_proc_name_hash", "f_parent_proc_hash", "f_proc_cwd_hash", "f_lineage_bag_hash",
    "f_parent_child_pair_hash", "f_root_ancestor_basename_hash", "f_process_tree_id_hash",
)

_W_OFF = {}
_off = 0
for _n, _c, _d in _EMBED:
    _W_OFF[_n] = _off
    _off += _d
_W_FLOAT_OFF = _off

_SMALL = [(n, c, d) for (n, c, d) in _EMBED if n not in _BIG_NAMES]
_OH_OFF, _SDIM_OFF = [], []
_o1 = _o2 = 0
for _n, _c, _d in _SMALL:
    _OH_OFF.append(_o1)
    _SDIM_OFF.append(_o2)
    _o1 += _c
    _o2 += _d
_OH_TOT = _o1
_SDIM_TOT = _o2
_OH_PAD = 288
_IDX_PAD = 32

_BIG = []
_o3 = 0
for _n in _BIG_NAMES:
    _c, _d = next((c, d) for (nm, c, d) in _EMBED if nm == _n)
    _BIG.append((_n, _c, _d, _d, _o3))
    _o3 += _d
_RAW_COLS = _o3

_TILE = 512
_LANES = 128
_GROUPS = _N // _LANES


def _fuse_body(bd_ref, ws_ref, t_ref):
    t_ref[...] = jnp.dot(bd_ref[...], ws_ref[...], preferred_element_type=jnp.float32)


_DN0 = (((0,), (0,)), ((), ()))


def _tc_body(idx_ref, r0, r1, r2, r3, r4, r5, r6, fl_ref, s_ref, tgt_ref,
             inv_ref, t_ref, wb_ref, wf_ref, b_ref, out_ref):
    idxf = idx_ref[...].astype(jnp.float32)
    g = lax.dot_general(idxf, s_ref[...], _DN0,
                        preferred_element_type=jnp.float32)
    oh = jnp.where(jnp.abs(g - tgt_ref[...]) < 0.5, 1.0, 0.0)
    raw = jnp.concatenate(
        [r[...] for r in (r0, r1, r2, r3, r4, r5, r6)], axis=1)
    fl = fl_ref[...] * inv_ref[...]
    acc = jnp.dot(oh, t_ref[...], preferred_element_type=jnp.float32)
    acc = acc + jnp.dot(raw, wb_ref[...], preferred_element_type=jnp.float32)
    acc = acc + lax.dot_general(fl, wf_ref[...], _DN0,
                                preferred_element_type=jnp.float32)
    out_ref[...] = acc + b_ref[...]


@functools.lru_cache(maxsize=1)
def _make_sc_gather():
    info = plsc.get_sparse_core_info()
    nc, ns = info.num_cores, info.num_subcores
    nw = nc * ns
    tpw = _N // nw
    rpw = _B // nw
    ch_r = 4
    half = ch_r * _L
    n_bodies = rpw // (2 * ch_r)
    runs = ((0, 104), (104, 96))
    mesh = plsc.VectorSubcoreMesh(core_axis_name="c", subcore_axis_name="s")

    @functools.partial(
        pl.kernel, mesh=mesh,
        compiler_params=pltpu.CompilerParams(use_tc_tiling_on_sc=False),
        out_type=[jax.ShapeDtypeStruct((_N, w), jnp.float32)
                  for (_n, _c, _d, w, _o) in _BIG],
        scratch_types=[
            pltpu.VMEM((rpw, _L), jnp.int32),
            pltpu.VMEM((half, 32), jnp.float32),
            pltpu.VMEM((half, 32), jnp.float32),
            pltpu.VMEM((half, 16), jnp.float32),
            pltpu.VMEM((half, 16), jnp.float32),
            pltpu.VMEM((half, 8), jnp.float32),
            pltpu.VMEM((half, 8), jnp.float32),
            pltpu.SemaphoreType.DMA,
            pltpu.SemaphoreType.DMA,
        ],
    )
    def sc_gather(t0, t1, t2, t3, t4, t5, t6, i0, i1, i2, i3, i4, i5, i6,
                  o0, o1, o2, o3, o4, o5, o6,
                  idx_v, ra32, rb32, ra16, rb16, ra8, rb8, sem_g, sem_o):
        wid = lax.axis_index("s") * nc + lax.axis_index("c")
        tabs = (t0, t1, t2, t3, t4, t5, t6)
        idxs = (i0, i1, i2, i3, i4, i5, i6)
        outs = (o0, o1, o2, o3, o4, o5, o6)
        row0 = wid * rpw
        tok0 = row0 * _L
        for f, (_nm, _card, _dim, w, _col) in enumerate(_BIG):
            tab, idxh, outh = tabs[f], idxs[f], outs[f]
            ra, rb = {32: (ra32, rb32), 16: (ra16, rb16), 8: (ra8, rb8)}[w]
            pltpu.sync_copy(idxh.at[pl.ds(row0, rpw)], idx_v)

            def fire_half(r_base, buf, tab=tab):
                cps = []
                for r2 in range(ch_r):
                    for (c0, cl) in runs:
                        cps.append(pltpu.async_copy(
                            tab.at[idx_v.at[r_base + r2, pl.ds(c0, cl)]],
                            buf.at[pl.ds(r2 * _L + c0, cl)],
                            sem_g,
                        ))
                return cps

            def body(k, carry, tab=tab, outh=outh, ra=ra, rb=rb,
                     fire_half=fire_half):
                ra_row = k * 2 * ch_r
                rb_row = ra_row + ch_r
                cps_a = fire_half(ra_row, ra)
                cps_b = fire_half(rb_row, rb)
                for cp in cps_a:
                    cp.wait()
                st_a = pltpu.async_copy(
                    ra, outh.at[pl.ds(tok0 + ra_row * _L, half)], sem_o)
                for cp in cps_b:
                    cp.wait()
                st_b = pltpu.async_copy(
                    rb, outh.at[pl.ds(tok0 + rb_row * _L, half)], sem_o)
                st_a.wait()
                st_b.wait()
                return carry

            lax.fori_loop(0, n_bodies, body, 0)

    return sc_gather


def kernel(f_event_type, emb_f_event_type, f_kprobe_function, emb_f_kprobe_function, f_kprobe_policy, emb_f_kprobe_policy, f_kprobe_action, emb_f_kprobe_action, f_proc_uid_bucket, emb_f_proc_uid_bucket, f_dst_port_bucket, emb_f_dst_port_bucket, f_args_length_bucket, emb_f_args_length_bucket, f_cap_count_bucket, emb_f_cap_count_bucket, f_path_sens_cwd, emb_f_path_sens_cwd, f_path_sens_binary, emb_f_path_sens_binary, f_path_sens_kp, emb_f_path_sens_kp, f_proc_name_hash, emb_f_proc_name_hash, f_parent_proc_hash, emb_f_parent_proc_hash, f_proc_cwd_hash, emb_f_proc_cwd_hash, f_lineage_bag_hash, emb_f_lineage_bag_hash, f_cmdline_entropy, emb_f_cmdline_entropy, f_cmdline_compress, emb_f_cmdline_compress, f_time_since_parent_exec, emb_f_time_since_parent_exec, f_kp_fd_install_path_sensitivity, emb_f_kp_fd_install_path_sensitivity, f_kp_mmap_path_sensitivity, emb_f_kp_mmap_path_sensitivity, f_kp_tcp_connect_dst_port_bucket, emb_f_kp_tcp_connect_dst_port_bucket, f_kp_tcp_connect_sock_family, emb_f_kp_tcp_connect_sock_family, f_action_family, emb_f_action_family, f_lineage_depth, emb_f_lineage_depth, f_parent_child_pair_hash, emb_f_parent_child_pair_hash, f_root_ancestor_basename_hash, emb_f_root_ancestor_basename_hash, f_process_tree_id_hash, emb_f_process_tree_id_hash, f_delta_t_log_bucket, emb_f_delta_t_log_bucket, f_process_age_log_bucket, emb_f_process_age_log_bucket, f_path_category, emb_f_path_category, f_dst_ip_category, emb_f_dst_ip_category, f_dst_port_category, emb_f_dst_port_category, f_object_category, emb_f_object_category, f_is_procfs_walk, f_uid_eq_parent, f_is_setuid_exec, f_kp_commit_creds_uid_change, f_kp_commit_creds_cap_change, f_kp_udp_sendmsg_dport_eq_53, f_kp_fd_install_fd_int32, f_kp_mmap_prot_uint, f_kp_mprotect_prot_uint, proj_W, proj_b):
    inp = dict(locals())

    idx_cols = [inp[n].reshape(_N).astype(jnp.int32) for (n, _c, _d) in _SMALL]
    idx_mat = jnp.stack(idx_cols, axis=0)
    idx_mat = jnp.pad(idx_mat, ((0, _IDX_PAD - len(_SMALL)), (0, 0)))

    fl_cols = [inp[n].reshape(_N).astype(jnp.float32) for (n, _s) in _FLOATS]
    fl_mat = jnp.stack(fl_cols, axis=0)
    fl_mat = jnp.pad(fl_mat, ((0, 16 - len(_FLOATS)), (0, 0)))

    big_idx = [inp[n].astype(jnp.int32) for (n, _c, _d, _w, _o) in _BIG]
    big_tab = [inp["emb_" + n] for (n, _c, _d, _w, _o) in _BIG]

    bd = jnp.zeros((_OH_PAD, _SDIM_TOT), jnp.float32)
    for (n, c, d), oho, sdo in zip(_SMALL, _OH_OFF, _SDIM_OFF):
        bd = bd.at[oho:oho + c, sdo:sdo + d].set(inp["emb_" + n])
    ws = jnp.concatenate(
        [proj_W[_W_OFF[n]:_W_OFF[n] + d] for (n, _c, d) in _SMALL], axis=0)
    wb = jnp.concatenate(
        [proj_W[_W_OFF[n]:_W_OFF[n] + d] for (n, _c, d, _w, _o) in _BIG], axis=0)
    wf = jnp.concatenate(
        [proj_W[_W_FLOAT_OFF:_W_FLOAT_OFF + len(_FLOATS)],
         jnp.zeros((16 - len(_FLOATS), _D), jnp.float32)], axis=0)
    b2 = proj_b.reshape(1, _D)

    s_np = np.zeros((_IDX_PAD, _OH_PAD), np.float32)
    tgt_np = np.full((1, _OH_PAD), -1.0, np.float32)
    for f, ((n, c, d), oho) in enumerate(zip(_SMALL, _OH_OFF)):
        s_np[f, oho:oho + c] = 1.0
        tgt_np[0, oho:oho + c] = np.arange(c, dtype=np.float32)
    inv_np = np.ones((16, 1), np.float32)
    for f, (n, sc) in enumerate(_FLOATS):
        inv_np[f, 0] = 1.0 / sc
    s_c = jnp.asarray(s_np)
    tgt_c = jnp.asarray(tgt_np)
    inv_c = jnp.asarray(inv_np)

    t_small = pl.pallas_call(
        _fuse_body,
        out_shape=jax.ShapeDtypeStruct((_OH_PAD, _D), jnp.float32),
    )(bd, ws)

    raws = _make_sc_gather()(*big_tab, *big_idx)

    out = pl.pallas_call(
        _tc_body,
        grid=(_N // _TILE,),
        in_specs=[
            pl.BlockSpec((_IDX_PAD, _TILE), lambda i: (0, i)),
        ] + [
            pl.BlockSpec((_TILE, w), lambda i: (i, 0))
            for (_n2, _c2, _d2, w, _o2) in _BIG
        ] + [
            pl.BlockSpec((16, _TILE), lambda i: (0, i)),
            pl.BlockSpec((_IDX_PAD, _OH_PAD), lambda i: (0, 0)),
            pl.BlockSpec((1, _OH_PAD), lambda i: (0, 0)),
            pl.BlockSpec((16, 1), lambda i: (0, 0)),
            pl.BlockSpec((_OH_PAD, _D), lambda i: (0, 0)),
            pl.BlockSpec((_RAW_COLS, _D), lambda i: (0, 0)),
            pl.BlockSpec((16, _D), lambda i: (0, 0)),
            pl.BlockSpec((1, _D), lambda i: (0, 0)),
        ],
        out_specs=pl.BlockSpec((_TILE, _D), lambda i: (i, 0)),
        out_shape=jax.ShapeDtypeStruct((_N, _D), jnp.float32),
    )(idx_mat, *raws, fl_mat, s_c, tgt_c, inv_c, t_small, wb, wf, b2)

    return out.reshape(_B, _L, _D)


# speedup vs baseline: 1.8958x; 1.7418x over previous
import functools

import numpy as np
import jax
import jax.numpy as jnp
from jax import lax
from jax.experimental import pallas as pl
from jax.experimental.pallas import tpu as pltpu
from jax.experimental.pallas import tpu_sc as plsc

_B, _L = 1024, 200
_N = _B * _L
_D = 128

_EMBED = [
    ("f_event_type", 32, 8), ("f_kprobe_function", 64, 16), ("f_kprobe_policy", 8, 8),
    ("f_kprobe_action", 8, 4), ("f_proc_uid_bucket", 8, 4), ("f_dst_port_bucket", 7, 4),
    ("f_args_length_bucket", 6, 4), ("f_cap_count_bucket", 5, 4), ("f_path_sens_cwd", 4, 8),
    ("f_path_sens_binary", 4, 8), ("f_path_sens_kp", 4, 8), ("f_proc_name_hash", 65536, 32),
    ("f_parent_proc_hash", 65536, 32), ("f_proc_cwd_hash", 16384, 16), ("f_lineage_bag_hash", 16384, 16),
    ("f_cmdline_entropy", 4, 4), ("f_cmdline_compress", 4, 4), ("f_time_since_parent_exec", 8, 4),
    ("f_kp_fd_install_path_sensitivity", 4, 4), ("f_kp_mmap_path_sensitivity", 4, 4),
    ("f_kp_tcp_connect_dst_port_bucket", 7, 4), ("f_kp_tcp_connect_sock_family", 8, 4),
    ("f_action_family", 16, 8), ("f_lineage_depth", 16, 4), ("f_parent_child_pair_hash", 1024, 16),
    ("f_root_ancestor_basename_hash", 1024, 8), ("f_process_tree_id_hash", 4096, 8),
    ("f_delta_t_log_bucket", 10, 4), ("f_process_age_log_bucket", 10, 4), ("f_path_category", 16, 4),
    ("f_dst_ip_category", 8, 4), ("f_dst_port_category", 8, 4), ("f_object_category", 8, 4),
]
_FLOATS = [
    ("f_is_procfs_walk", 1.0), ("f_uid_eq_parent", 1.0), ("f_is_setuid_exec", 1.0),
    ("f_kp_commit_creds_uid_change", 1.0), ("f_kp_commit_creds_cap_change", 1.0),
    ("f_kp_udp_sendmsg_dport_eq_53", 1.0), ("f_kp_fd_install_fd_int32", 1024.0),
    ("f_kp_mmap_prot_uint", 256.0), ("f_kp_mprotect_prot_uint", 256.0),
]
_BIG_NAMES = (
    "f_proc_name_hash", "f_parent_proc_hash", "f_proc_cwd_hash", "f_lineage_bag_hash",
    "f_parent_child_pair_hash", "f_root_ancestor_basename_hash", "f_process_tree_id_hash",
)

_W_OFF = {}
_off = 0
for _n, _c, _d in _EMBED:
    _W_OFF[_n] = _off
    _off += _d
_W_FLOAT_OFF = _off

_SMALL = [(n, c, d) for (n, c, d) in _EMBED if n not in _BIG_NAMES]
_OH_OFF, _SDIM_OFF = [], []
_o1 = _o2 = 0
for _n, _c, _d in _SMALL:
    _OH_OFF.append(_o1)
    _SDIM_OFF.append(_o2)
    _o1 += _c
    _o2 += _d
_OH_TOT = _o1
_SDIM_TOT = _o2
_OH_PAD = 288
_IDX_PAD = 32

_BIG = []
_o3 = 0
for _n in _BIG_NAMES:
    _c, _d = next((c, d) for (nm, c, d) in _EMBED if nm == _n)
    _BIG.append((_n, _c, _d, _d, _o3))
    _o3 += _d
_RAW_COLS = _o3

_TILE = 512
_LANES = 128
_GROUPS = _N // _LANES


def _fuse_body(bd_ref, ws_ref, t_ref):
    t_ref[...] = jnp.dot(bd_ref[...], ws_ref[...], preferred_element_type=jnp.float32)


_DN0 = (((0,), (0,)), ((), ()))


def _tc_body(idx_ref, raw_ref, fl_ref, s_ref, tgt_ref,
             inv_ref, t_ref, wb_ref, wf_ref, b_ref, out_ref):
    idxf = idx_ref[...].astype(jnp.float32)
    g = lax.dot_general(idxf, s_ref[...], _DN0,
                        preferred_element_type=jnp.float32)
    oh = jnp.where(jnp.abs(g - tgt_ref[...]) < 0.5, 1.0, 0.0)
    fl = fl_ref[...] * inv_ref[...]
    acc = jnp.dot(oh, t_ref[...], preferred_element_type=jnp.float32)
    acc = acc + jnp.dot(raw_ref[...], wb_ref[...],
                        preferred_element_type=jnp.float32)
    acc = acc + lax.dot_general(fl, wf_ref[...], _DN0,
                                preferred_element_type=jnp.float32)
    out_ref[...] = acc + b_ref[...]


@functools.lru_cache(maxsize=1)
def _make_sc_gather():
    info = plsc.get_sparse_core_info()
    nc, ns = info.num_cores, info.num_subcores
    nw = nc * ns
    tpw = _N // nw
    rpw = _B // nw
    ch_r = 4
    half = ch_r * _L
    n_bodies = rpw // (2 * ch_r)
    runs = ((0, 104), (104, 96))
    mesh = plsc.VectorSubcoreMesh(core_axis_name="c", subcore_axis_name="s")

    @functools.partial(
        pl.kernel, mesh=mesh,
        compiler_params=pltpu.CompilerParams(use_tc_tiling_on_sc=False),
        out_type=jax.ShapeDtypeStruct((_N, _RAW_COLS), jnp.float32),
        scratch_types=[
            pltpu.VMEM((rpw, _L), jnp.int32),
            pltpu.VMEM((half, 32), jnp.float32),
            pltpu.VMEM((half, 32), jnp.float32),
            pltpu.VMEM((half, 16), jnp.float32),
            pltpu.VMEM((half, 16), jnp.float32),
            pltpu.VMEM((half, 8), jnp.float32),
            pltpu.VMEM((half, 8), jnp.float32),
            pltpu.SemaphoreType.DMA,
            pltpu.SemaphoreType.DMA,
        ],
    )
    def sc_gather(t0, t1, t2, t3, t4, t5, t6, i0, i1, i2, i3, i4, i5, i6,
                  out_ref, idx_v, ra32, rb32, ra16, rb16, ra8, rb8,
                  sem_g, sem_o):
        wid = lax.axis_index("s") * nc + lax.axis_index("c")
        tabs = (t0, t1, t2, t3, t4, t5, t6)
        idxs = (i0, i1, i2, i3, i4, i5, i6)
        row0 = wid * rpw
        tok0 = row0 * _L
        for f, (_nm, _card, _dim, w, col) in enumerate(_BIG):
            tab, idxh = tabs[f], idxs[f]
            ra, rb = {32: (ra32, rb32), 16: (ra16, rb16), 8: (ra8, rb8)}[w]
            pltpu.sync_copy(idxh.at[pl.ds(row0, rpw)], idx_v)

            def fire_half(r_base, buf, tab=tab):
                cps = []
                for r2 in range(ch_r):
                    for (c0, cl) in runs:
                        cps.append(pltpu.async_copy(
                            tab.at[idx_v.at[r_base + r2, pl.ds(c0, cl)]],
                            buf.at[pl.ds(r2 * _L + c0, cl)],
                            sem_g,
                        ))
                return cps

            def body(k, carry, tab=tab, ra=ra, rb=rb, col=col, w=w,
                     fire_half=fire_half):
                ra_row = k * 2 * ch_r
                rb_row = ra_row + ch_r
                cps_a = fire_half(ra_row, ra)
                cps_b = fire_half(rb_row, rb)
                for cp in cps_a:
                    cp.wait()
                st_a = pltpu.async_copy(
                    ra,
                    out_ref.at[pl.ds(tok0 + ra_row * _L, half), pl.ds(col, w)],
                    sem_o)
                for cp in cps_b:
                    cp.wait()
                st_b = pltpu.async_copy(
                    rb,
                    out_ref.at[pl.ds(tok0 + rb_row * _L, half), pl.ds(col, w)],
                    sem_o)
                st_a.wait()
                st_b.wait()
                return carry

            lax.fori_loop(0, n_bodies, body, 0)

    return sc_gather


def kernel(f_event_type, emb_f_event_type, f_kprobe_function, emb_f_kprobe_function, f_kprobe_policy, emb_f_kprobe_policy, f_kprobe_action, emb_f_kprobe_action, f_proc_uid_bucket, emb_f_proc_uid_bucket, f_dst_port_bucket, emb_f_dst_port_bucket, f_args_length_bucket, emb_f_args_length_bucket, f_cap_count_bucket, emb_f_cap_count_bucket, f_path_sens_cwd, emb_f_path_sens_cwd, f_path_sens_binary, emb_f_path_sens_binary, f_path_sens_kp, emb_f_path_sens_kp, f_proc_name_hash, emb_f_proc_name_hash, f_parent_proc_hash, emb_f_parent_proc_hash, f_proc_cwd_hash, emb_f_proc_cwd_hash, f_lineage_bag_hash, emb_f_lineage_bag_hash, f_cmdline_entropy, emb_f_cmdline_entropy, f_cmdline_compress, emb_f_cmdline_compress, f_time_since_parent_exec, emb_f_time_since_parent_exec, f_kp_fd_install_path_sensitivity, emb_f_kp_fd_install_path_sensitivity, f_kp_mmap_path_sensitivity, emb_f_kp_mmap_path_sensitivity, f_kp_tcp_connect_dst_port_bucket, emb_f_kp_tcp_connect_dst_port_bucket, f_kp_tcp_connect_sock_family, emb_f_kp_tcp_connect_sock_family, f_action_family, emb_f_action_family, f_lineage_depth, emb_f_lineage_depth, f_parent_child_pair_hash, emb_f_parent_child_pair_hash, f_root_ancestor_basename_hash, emb_f_root_ancestor_basename_hash, f_process_tree_id_hash, emb_f_process_tree_id_hash, f_delta_t_log_bucket, emb_f_delta_t_log_bucket, f_process_age_log_bucket, emb_f_process_age_log_bucket, f_path_category, emb_f_path_category, f_dst_ip_category, emb_f_dst_ip_category, f_dst_port_category, emb_f_dst_port_category, f_object_category, emb_f_object_category, f_is_procfs_walk, f_uid_eq_parent, f_is_setuid_exec, f_kp_commit_creds_uid_change, f_kp_commit_creds_cap_change, f_kp_udp_sendmsg_dport_eq_53, f_kp_fd_install_fd_int32, f_kp_mmap_prot_uint, f_kp_mprotect_prot_uint, proj_W, proj_b):
    inp = dict(locals())

    idx_cols = [inp[n].reshape(_N).astype(jnp.int32) for (n, _c, _d) in _SMALL]
    idx_mat = jnp.stack(idx_cols, axis=0)
    idx_mat = jnp.pad(idx_mat, ((0, _IDX_PAD - len(_SMALL)), (0, 0)))

    fl_cols = [inp[n].reshape(_N).astype(jnp.float32) for (n, _s) in _FLOATS]
    fl_mat = jnp.stack(fl_cols, axis=0)
    fl_mat = jnp.pad(fl_mat, ((0, 16 - len(_FLOATS)), (0, 0)))

    big_idx = [inp[n].astype(jnp.int32) for (n, _c, _d, _w, _o) in _BIG]
    big_tab = [inp["emb_" + n] for (n, _c, _d, _w, _o) in _BIG]

    bd = jnp.zeros((_OH_PAD, _SDIM_TOT), jnp.float32)
    for (n, c, d), oho, sdo in zip(_SMALL, _OH_OFF, _SDIM_OFF):
        bd = bd.at[oho:oho + c, sdo:sdo + d].set(inp["emb_" + n])
    ws = jnp.concatenate(
        [proj_W[_W_OFF[n]:_W_OFF[n] + d] for (n, _c, d) in _SMALL], axis=0)
    wb = jnp.concatenate(
        [proj_W[_W_OFF[n]:_W_OFF[n] + d] for (n, _c, d, _w, _o) in _BIG], axis=0)
    wf = jnp.concatenate(
        [proj_W[_W_FLOAT_OFF:_W_FLOAT_OFF + len(_FLOATS)],
         jnp.zeros((16 - len(_FLOATS), _D), jnp.float32)], axis=0)
    b2 = proj_b.reshape(1, _D)

    s_np = np.zeros((_IDX_PAD, _OH_PAD), np.float32)
    tgt_np = np.full((1, _OH_PAD), -1.0, np.float32)
    for f, ((n, c, d), oho) in enumerate(zip(_SMALL, _OH_OFF)):
        s_np[f, oho:oho + c] = 1.0
        tgt_np[0, oho:oho + c] = np.arange(c, dtype=np.float32)
    inv_np = np.ones((16, 1), np.float32)
    for f, (n, sc) in enumerate(_FLOATS):
        inv_np[f, 0] = 1.0 / sc
    s_c = jnp.asarray(s_np)
    tgt_c = jnp.asarray(tgt_np)
    inv_c = jnp.asarray(inv_np)

    t_small = pl.pallas_call(
        _fuse_body,
        out_shape=jax.ShapeDtypeStruct((_OH_PAD, _D), jnp.float32),
    )(bd, ws)

    raw = _make_sc_gather()(*big_tab, *big_idx)

    out = pl.pallas_call(
        _tc_body,
        grid=(_N // _TILE,),
        in_specs=[
            pl.BlockSpec((_IDX_PAD, _TILE), lambda i: (0, i)),
            pl.BlockSpec((_TILE, _RAW_COLS), lambda i: (i, 0)),
            pl.BlockSpec((16, _TILE), lambda i: (0, i)),
            pl.BlockSpec((_IDX_PAD, _OH_PAD), lambda i: (0, 0)),
            pl.BlockSpec((1, _OH_PAD), lambda i: (0, 0)),
            pl.BlockSpec((16, 1), lambda i: (0, 0)),
            pl.BlockSpec((_OH_PAD, _D), lambda i: (0, 0)),
            pl.BlockSpec((_RAW_COLS, _D), lambda i: (0, 0)),
            pl.BlockSpec((16, _D), lambda i: (0, 0)),
            pl.BlockSpec((1, _D), lambda i: (0, 0)),
        ],
        out_specs=pl.BlockSpec((_TILE, _D), lambda i: (i, 0)),
        out_shape=jax.ShapeDtypeStruct((_N, _D), jnp.float32),
    )(idx_mat, raw, fl_mat, s_c, tgt_c, inv_c, t_small, wb, wf, b2)

    return out.reshape(_B, _L, _D)


# speedup vs baseline: 2.2082x; 1.1648x over previous
import functools

import numpy as np
import jax
import jax.numpy as jnp
from jax import lax
from jax.experimental import pallas as pl
from jax.experimental.pallas import tpu as pltpu
from jax.experimental.pallas import tpu_sc as plsc

_B, _L = 1024, 200
_N = _B * _L
_D = 128

_EMBED = [
    ("f_event_type", 32, 8), ("f_kprobe_function", 64, 16), ("f_kprobe_policy", 8, 8),
    ("f_kprobe_action", 8, 4), ("f_proc_uid_bucket", 8, 4), ("f_dst_port_bucket", 7, 4),
    ("f_args_length_bucket", 6, 4), ("f_cap_count_bucket", 5, 4), ("f_path_sens_cwd", 4, 8),
    ("f_path_sens_binary", 4, 8), ("f_path_sens_kp", 4, 8), ("f_proc_name_hash", 65536, 32),
    ("f_parent_proc_hash", 65536, 32), ("f_proc_cwd_hash", 16384, 16), ("f_lineage_bag_hash", 16384, 16),
    ("f_cmdline_entropy", 4, 4), ("f_cmdline_compress", 4, 4), ("f_time_since_parent_exec", 8, 4),
    ("f_kp_fd_install_path_sensitivity", 4, 4), ("f_kp_mmap_path_sensitivity", 4, 4),
    ("f_kp_tcp_connect_dst_port_bucket", 7, 4), ("f_kp_tcp_connect_sock_family", 8, 4),
    ("f_action_family", 16, 8), ("f_lineage_depth", 16, 4), ("f_parent_child_pair_hash", 1024, 16),
    ("f_root_ancestor_basename_hash", 1024, 8), ("f_process_tree_id_hash", 4096, 8),
    ("f_delta_t_log_bucket", 10, 4), ("f_process_age_log_bucket", 10, 4), ("f_path_category", 16, 4),
    ("f_dst_ip_category", 8, 4), ("f_dst_port_category", 8, 4), ("f_object_category", 8, 4),
]
_FLOATS = [
    ("f_is_procfs_walk", 1.0), ("f_uid_eq_parent", 1.0), ("f_is_setuid_exec", 1.0),
    ("f_kp_commit_creds_uid_change", 1.0), ("f_kp_commit_creds_cap_change", 1.0),
    ("f_kp_udp_sendmsg_dport_eq_53", 1.0), ("f_kp_fd_install_fd_int32", 1024.0),
    ("f_kp_mmap_prot_uint", 256.0), ("f_kp_mprotect_prot_uint", 256.0),
]
_BIG_NAMES = (
    "f_proc_name_hash", "f_parent_proc_hash", "f_proc_cwd_hash", "f_lineage_bag_hash",
    "f_parent_child_pair_hash", "f_root_ancestor_basename_hash", "f_process_tree_id_hash",
)

_W_OFF = {}
_off = 0
for _n, _c, _d in _EMBED:
    _W_OFF[_n] = _off
    _off += _d
_W_FLOAT_OFF = _off

_SMALL = [(n, c, d) for (n, c, d) in _EMBED if n not in _BIG_NAMES]
_OH_OFF, _SDIM_OFF = [], []
_o1 = _o2 = 0
for _n, _c, _d in _SMALL:
    _OH_OFF.append(_o1)
    _SDIM_OFF.append(_o2)
    _o1 += _c
    _o2 += _d
_OH_TOT = _o1
_SDIM_TOT = _o2
_OH_PAD = 288
_IDX_PAD = 32

_BIG = []
_o3 = 0
for _n in _BIG_NAMES:
    _c, _d = next((c, d) for (nm, c, d) in _EMBED if nm == _n)
    _BIG.append((_n, _c, _d, _d, _o3))
    _o3 += _d
_RAW_COLS = _o3

_TILE = 1024
_LANES = 128
_GROUPS = _N // _LANES


def _fuse_body(bd_ref, ws_ref, t_ref):
    t_ref[...] = jnp.dot(bd_ref[...], ws_ref[...], preferred_element_type=jnp.float32)


_DN0 = (((0,), (0,)), ((), ()))


def _tc_body(idx_ref, raw_ref, fl_ref, s_ref, tgt_ref,
             inv_ref, t_ref, wb_ref, wf_ref, b_ref, out_ref):
    idxf = idx_ref[...].astype(jnp.float32)
    g = lax.dot_general(idxf, s_ref[...], _DN0,
                        preferred_element_type=jnp.float32)
    oh = jnp.where(jnp.abs(g - tgt_ref[...]) < 0.5, 1.0, 0.0)
    fl = fl_ref[...] * inv_ref[...]
    acc = jnp.dot(oh, t_ref[...], preferred_element_type=jnp.float32)
    acc = acc + jnp.dot(raw_ref[...], wb_ref[...],
                        preferred_element_type=jnp.float32)
    acc = acc + lax.dot_general(fl, wf_ref[...], _DN0,
                                preferred_element_type=jnp.float32)
    out_ref[...] = acc + b_ref[...]


@functools.lru_cache(maxsize=1)
def _make_sc_gather():
    info = plsc.get_sparse_core_info()
    nc, ns = info.num_cores, info.num_subcores
    nw = nc * ns
    tpw = _N // nw
    rpw = _B // nw
    ch_r = 4
    half = ch_r * _L
    n_bodies = rpw // (2 * ch_r)
    runs = ((0, 104), (104, 96))
    mesh = plsc.VectorSubcoreMesh(core_axis_name="c", subcore_axis_name="s")

    @functools.partial(
        pl.kernel, mesh=mesh,
        compiler_params=pltpu.CompilerParams(use_tc_tiling_on_sc=False),
        out_type=jax.ShapeDtypeStruct((_N, _RAW_COLS), jnp.float32),
        scratch_types=[
            pltpu.VMEM((rpw, _L), jnp.int32),
            pltpu.VMEM((half, 32), jnp.float32),
            pltpu.VMEM((half, 32), jnp.float32),
            pltpu.VMEM((half, 16), jnp.float32),
            pltpu.VMEM((half, 16), jnp.float32),
            pltpu.VMEM((half, 8), jnp.float32),
            pltpu.VMEM((half, 8), jnp.float32),
            pltpu.SemaphoreType.DMA,
            pltpu.SemaphoreType.DMA,
        ],
    )
    def sc_gather(t0, t1, t2, t3, t4, t5, t6, i0, i1, i2, i3, i4, i5, i6,
                  out_ref, idx_v, ra32, rb32, ra16, rb16, ra8, rb8,
                  sem_g, sem_o):
        wid = lax.axis_index("s") * nc + lax.axis_index("c")
        tabs = (t0, t1, t2, t3, t4, t5, t6)
        idxs = (i0, i1, i2, i3, i4, i5, i6)
        row0 = wid * rpw
        tok0 = row0 * _L
        for f, (_nm, _card, _dim, w, col) in enumerate(_BIG):
            tab, idxh = tabs[f], idxs[f]
            ra, rb = {32: (ra32, rb32), 16: (ra16, rb16), 8: (ra8, rb8)}[w]
            pltpu.sync_copy(idxh.at[pl.ds(row0, rpw)], idx_v)

            def fire_half(r_base, buf, tab=tab):
                cps = []
                for r2 in range(ch_r):
                    for (c0, cl) in runs:
                        cps.append(pltpu.async_copy(
                            tab.at[idx_v.at[r_base + r2, pl.ds(c0, cl)]],
                            buf.at[pl.ds(r2 * _L + c0, cl)],
                            sem_g,
                        ))
                return cps

            def body(k, carry, tab=tab, ra=ra, rb=rb, col=col, w=w,
                     fire_half=fire_half):
                ra_row = k * 2 * ch_r
                rb_row = ra_row + ch_r
                cps_a = fire_half(ra_row, ra)
                cps_b = fire_half(rb_row, rb)
                for cp in cps_a:
                    cp.wait()
                st_a = pltpu.async_copy(
                    ra,
                    out_ref.at[pl.ds(tok0 + ra_row * _L, half), pl.ds(col, w)],
                    sem_o)
                for cp in cps_b:
                    cp.wait()
                st_b = pltpu.async_copy(
                    rb,
                    out_ref.at[pl.ds(tok0 + rb_row * _L, half), pl.ds(col, w)],
                    sem_o)
                st_a.wait()
                st_b.wait()
                return carry

            lax.fori_loop(0, n_bodies, body, 0)

    return sc_gather


def kernel(f_event_type, emb_f_event_type, f_kprobe_function, emb_f_kprobe_function, f_kprobe_policy, emb_f_kprobe_policy, f_kprobe_action, emb_f_kprobe_action, f_proc_uid_bucket, emb_f_proc_uid_bucket, f_dst_port_bucket, emb_f_dst_port_bucket, f_args_length_bucket, emb_f_args_length_bucket, f_cap_count_bucket, emb_f_cap_count_bucket, f_path_sens_cwd, emb_f_path_sens_cwd, f_path_sens_binary, emb_f_path_sens_binary, f_path_sens_kp, emb_f_path_sens_kp, f_proc_name_hash, emb_f_proc_name_hash, f_parent_proc_hash, emb_f_parent_proc_hash, f_proc_cwd_hash, emb_f_proc_cwd_hash, f_lineage_bag_hash, emb_f_lineage_bag_hash, f_cmdline_entropy, emb_f_cmdline_entropy, f_cmdline_compress, emb_f_cmdline_compress, f_time_since_parent_exec, emb_f_time_since_parent_exec, f_kp_fd_install_path_sensitivity, emb_f_kp_fd_install_path_sensitivity, f_kp_mmap_path_sensitivity, emb_f_kp_mmap_path_sensitivity, f_kp_tcp_connect_dst_port_bucket, emb_f_kp_tcp_connect_dst_port_bucket, f_kp_tcp_connect_sock_family, emb_f_kp_tcp_connect_sock_family, f_action_family, emb_f_action_family, f_lineage_depth, emb_f_lineage_depth, f_parent_child_pair_hash, emb_f_parent_child_pair_hash, f_root_ancestor_basename_hash, emb_f_root_ancestor_basename_hash, f_process_tree_id_hash, emb_f_process_tree_id_hash, f_delta_t_log_bucket, emb_f_delta_t_log_bucket, f_process_age_log_bucket, emb_f_process_age_log_bucket, f_path_category, emb_f_path_category, f_dst_ip_category, emb_f_dst_ip_category, f_dst_port_category, emb_f_dst_port_category, f_object_category, emb_f_object_category, f_is_procfs_walk, f_uid_eq_parent, f_is_setuid_exec, f_kp_commit_creds_uid_change, f_kp_commit_creds_cap_change, f_kp_udp_sendmsg_dport_eq_53, f_kp_fd_install_fd_int32, f_kp_mmap_prot_uint, f_kp_mprotect_prot_uint, proj_W, proj_b):
    inp = dict(locals())

    idx_cols = [inp[n].reshape(_N).astype(jnp.int32) for (n, _c, _d) in _SMALL]
    idx_mat = jnp.stack(idx_cols, axis=0)
    idx_mat = jnp.pad(idx_mat, ((0, _IDX_PAD - len(_SMALL)), (0, 0)))

    fl_cols = [inp[n].reshape(_N).astype(jnp.float32) for (n, _s) in _FLOATS]
    fl_mat = jnp.stack(fl_cols, axis=0)
    fl_mat = jnp.pad(fl_mat, ((0, 16 - len(_FLOATS)), (0, 0)))

    big_idx = [inp[n].astype(jnp.int32) for (n, _c, _d, _w, _o) in _BIG]
    big_tab = [inp["emb_" + n] for (n, _c, _d, _w, _o) in _BIG]

    bd = jnp.zeros((_OH_PAD, _SDIM_TOT), jnp.float32)
    for (n, c, d), oho, sdo in zip(_SMALL, _OH_OFF, _SDIM_OFF):
        bd = bd.at[oho:oho + c, sdo:sdo + d].set(inp["emb_" + n])
    ws = jnp.concatenate(
        [proj_W[_W_OFF[n]:_W_OFF[n] + d] for (n, _c, d) in _SMALL], axis=0)
    wb = jnp.concatenate(
        [proj_W[_W_OFF[n]:_W_OFF[n] + d] for (n, _c, d, _w, _o) in _BIG], axis=0)
    wf = jnp.concatenate(
        [proj_W[_W_FLOAT_OFF:_W_FLOAT_OFF + len(_FLOATS)],
         jnp.zeros((16 - len(_FLOATS), _D), jnp.float32)], axis=0)
    b2 = proj_b.reshape(1, _D)

    s_np = np.zeros((_IDX_PAD, _OH_PAD), np.float32)
    tgt_np = np.full((1, _OH_PAD), -1.0, np.float32)
    for f, ((n, c, d), oho) in enumerate(zip(_SMALL, _OH_OFF)):
        s_np[f, oho:oho + c] = 1.0
        tgt_np[0, oho:oho + c] = np.arange(c, dtype=np.float32)
    inv_np = np.ones((16, 1), np.float32)
    for f, (n, sc) in enumerate(_FLOATS):
        inv_np[f, 0] = 1.0 / sc
    s_c = jnp.asarray(s_np)
    tgt_c = jnp.asarray(tgt_np)
    inv_c = jnp.asarray(inv_np)

    t_small = pl.pallas_call(
        _fuse_body,
        out_shape=jax.ShapeDtypeStruct((_OH_PAD, _D), jnp.float32),
    )(bd, ws)

    raw = _make_sc_gather()(*big_tab, *big_idx)

    out = pl.pallas_call(
        _tc_body,
        grid=(_N // _TILE,),
        in_specs=[
            pl.BlockSpec((_IDX_PAD, _TILE), lambda i: (0, i)),
            pl.BlockSpec((_TILE, _RAW_COLS), lambda i: (i, 0)),
            pl.BlockSpec((16, _TILE), lambda i: (0, i)),
            pl.BlockSpec((_IDX_PAD, _OH_PAD), lambda i: (0, 0)),
            pl.BlockSpec((1, _OH_PAD), lambda i: (0, 0)),
            pl.BlockSpec((16, 1), lambda i: (0, 0)),
            pl.BlockSpec((_OH_PAD, _D), lambda i: (0, 0)),
            pl.BlockSpec((_RAW_COLS, _D), lambda i: (0, 0)),
            pl.BlockSpec((16, _D), lambda i: (0, 0)),
            pl.BlockSpec((1, _D), lambda i: (0, 0)),
        ],
        out_specs=pl.BlockSpec((_TILE, _D), lambda i: (i, 0)),
        out_shape=jax.ShapeDtypeStruct((_N, _D), jnp.float32),
    )(idx_mat, raw, fl_mat, s_c, tgt_c, inv_c, t_small, wb, wf, b2)

    return out.reshape(_B, _L, _D)


# speedup vs baseline: 2.4473x; 1.1083x over previous
import functools

import numpy as np
import jax
import jax.numpy as jnp
from jax import lax
from jax.experimental import pallas as pl
from jax.experimental.pallas import tpu as pltpu
from jax.experimental.pallas import tpu_sc as plsc

_B, _L = 1024, 200
_N = _B * _L
_D = 128

_EMBED = [
    ("f_event_type", 32, 8), ("f_kprobe_function", 64, 16), ("f_kprobe_policy", 8, 8),
    ("f_kprobe_action", 8, 4), ("f_proc_uid_bucket", 8, 4), ("f_dst_port_bucket", 7, 4),
    ("f_args_length_bucket", 6, 4), ("f_cap_count_bucket", 5, 4), ("f_path_sens_cwd", 4, 8),
    ("f_path_sens_binary", 4, 8), ("f_path_sens_kp", 4, 8), ("f_proc_name_hash", 65536, 32),
    ("f_parent_proc_hash", 65536, 32), ("f_proc_cwd_hash", 16384, 16), ("f_lineage_bag_hash", 16384, 16),
    ("f_cmdline_entropy", 4, 4), ("f_cmdline_compress", 4, 4), ("f_time_since_parent_exec", 8, 4),
    ("f_kp_fd_install_path_sensitivity", 4, 4), ("f_kp_mmap_path_sensitivity", 4, 4),
    ("f_kp_tcp_connect_dst_port_bucket", 7, 4), ("f_kp_tcp_connect_sock_family", 8, 4),
    ("f_action_family", 16, 8), ("f_lineage_depth", 16, 4), ("f_parent_child_pair_hash", 1024, 16),
    ("f_root_ancestor_basename_hash", 1024, 8), ("f_process_tree_id_hash", 4096, 8),
    ("f_delta_t_log_bucket", 10, 4), ("f_process_age_log_bucket", 10, 4), ("f_path_category", 16, 4),
    ("f_dst_ip_category", 8, 4), ("f_dst_port_category", 8, 4), ("f_object_category", 8, 4),
]
_FLOATS = [
    ("f_is_procfs_walk", 1.0), ("f_uid_eq_parent", 1.0), ("f_is_setuid_exec", 1.0),
    ("f_kp_commit_creds_uid_change", 1.0), ("f_kp_commit_creds_cap_change", 1.0),
    ("f_kp_udp_sendmsg_dport_eq_53", 1.0), ("f_kp_fd_install_fd_int32", 1024.0),
    ("f_kp_mmap_prot_uint", 256.0), ("f_kp_mprotect_prot_uint", 256.0),
]
_BIG_NAMES = (
    "f_proc_name_hash", "f_parent_proc_hash", "f_proc_cwd_hash", "f_lineage_bag_hash",
    "f_parent_child_pair_hash", "f_root_ancestor_basename_hash", "f_process_tree_id_hash",
)

_W_OFF = {}
_off = 0
for _n, _c, _d in _EMBED:
    _W_OFF[_n] = _off
    _off += _d
_W_FLOAT_OFF = _off

_SMALL = [(n, c, d) for (n, c, d) in _EMBED if n not in _BIG_NAMES]
_OH_OFF, _SDIM_OFF = [], []
_o1 = _o2 = 0
for _n, _c, _d in _SMALL:
    _OH_OFF.append(_o1)
    _SDIM_OFF.append(_o2)
    _o1 += _c
    _o2 += _d
_OH_TOT = _o1
_SDIM_TOT = _o2
_OH_PAD = 288
_IDX_PAD = 32

_BIG = []
_o3 = 0
for _n in _BIG_NAMES:
    _c, _d = next((c, d) for (nm, c, d) in _EMBED if nm == _n)
    _BIG.append((_n, _c, _d, _d, _o3))
    _o3 += _d
_RAW_COLS = _o3

_TILE = 2048
_LANES = 128
_GROUPS = _N // _LANES


def _fuse_body(bd_ref, ws_ref, t_ref):
    t_ref[...] = jnp.dot(bd_ref[...], ws_ref[...], preferred_element_type=jnp.float32)


_DN0 = (((0,), (0,)), ((), ()))


def _tc_body(idx_ref, raw_ref, fl_ref, s_ref, tgt_ref,
             inv_ref, t_ref, wb_ref, wf_ref, b_ref, out_ref):
    idxf = idx_ref[...].astype(jnp.float32)
    g = lax.dot_general(idxf, s_ref[...], _DN0,
                        preferred_element_type=jnp.float32)
    oh = jnp.where(jnp.abs(g - tgt_ref[...]) < 0.5, 1.0, 0.0)
    fl = fl_ref[...] * inv_ref[...]
    acc = jnp.dot(oh, t_ref[...], preferred_element_type=jnp.float32)
    acc = acc + jnp.dot(raw_ref[...], wb_ref[...],
                        preferred_element_type=jnp.float32)
    acc = acc + lax.dot_general(fl, wf_ref[...], _DN0,
                                preferred_element_type=jnp.float32)
    out_ref[...] = acc + b_ref[...]


@functools.lru_cache(maxsize=1)
def _make_sc_gather():
    info = plsc.get_sparse_core_info()
    nc, ns = info.num_cores, info.num_subcores
    nw = nc * ns
    tpw = _N // nw
    rpw = _B // nw
    ch_r = 4
    half = ch_r * _L
    n_bodies = rpw // (2 * ch_r)
    runs = ((0, 104), (104, 96))
    mesh = plsc.VectorSubcoreMesh(core_axis_name="c", subcore_axis_name="s")

    @functools.partial(
        pl.kernel, mesh=mesh,
        compiler_params=pltpu.CompilerParams(use_tc_tiling_on_sc=False),
        out_type=jax.ShapeDtypeStruct((_N, _RAW_COLS), jnp.float32),
        scratch_types=[
            pltpu.VMEM((rpw, _L), jnp.int32),
            pltpu.VMEM((half, 32), jnp.float32),
            pltpu.VMEM((half, 32), jnp.float32),
            pltpu.VMEM((half, 16), jnp.float32),
            pltpu.VMEM((half, 16), jnp.float32),
            pltpu.VMEM((half, 8), jnp.float32),
            pltpu.VMEM((half, 8), jnp.float32),
            pltpu.SemaphoreType.DMA,
            pltpu.SemaphoreType.DMA,
        ],
    )
    def sc_gather(t0, t1, t2, t3, t4, t5, t6, i0, i1, i2, i3, i4, i5, i6,
                  out_ref, idx_v, ra32, rb32, ra16, rb16, ra8, rb8,
                  sem_g, sem_o):
        wid = lax.axis_index("s") * nc + lax.axis_index("c")
        tabs = (t0, t1, t2, t3, t4, t5, t6)
        idxs = (i0, i1, i2, i3, i4, i5, i6)
        row0 = wid * rpw
        tok0 = row0 * _L
        for f, (_nm, _card, _dim, w, col) in enumerate(_BIG):
            tab, idxh = tabs[f], idxs[f]
            ra, rb = {32: (ra32, rb32), 16: (ra16, rb16), 8: (ra8, rb8)}[w]
            pltpu.sync_copy(idxh.at[pl.ds(row0, rpw)], idx_v)

            def fire_half(r_base, buf, tab=tab):
                cps = []
                for r2 in range(ch_r):
                    for (c0, cl) in runs:
                        cps.append(pltpu.async_copy(
                            tab.at[idx_v.at[r_base + r2, pl.ds(c0, cl)]],
                            buf.at[pl.ds(r2 * _L + c0, cl)],
                            sem_g,
                        ))
                return cps

            def body(k, carry, tab=tab, ra=ra, rb=rb, col=col, w=w,
                     fire_half=fire_half):
                ra_row = k * 2 * ch_r
                rb_row = ra_row + ch_r
                cps_a = fire_half(ra_row, ra)
                cps_b = fire_half(rb_row, rb)
                for cp in cps_a:
                    cp.wait()
                st_a = pltpu.async_copy(
                    ra,
                    out_ref.at[pl.ds(tok0 + ra_row * _L, half), pl.ds(col, w)],
                    sem_o)
                for cp in cps_b:
                    cp.wait()
                st_b = pltpu.async_copy(
                    rb,
                    out_ref.at[pl.ds(tok0 + rb_row * _L, half), pl.ds(col, w)],
                    sem_o)
                st_a.wait()
                st_b.wait()
                return carry

            lax.fori_loop(0, n_bodies, body, 0)

    return sc_gather


def kernel(f_event_type, emb_f_event_type, f_kprobe_function, emb_f_kprobe_function, f_kprobe_policy, emb_f_kprobe_policy, f_kprobe_action, emb_f_kprobe_action, f_proc_uid_bucket, emb_f_proc_uid_bucket, f_dst_port_bucket, emb_f_dst_port_bucket, f_args_length_bucket, emb_f_args_length_bucket, f_cap_count_bucket, emb_f_cap_count_bucket, f_path_sens_cwd, emb_f_path_sens_cwd, f_path_sens_binary, emb_f_path_sens_binary, f_path_sens_kp, emb_f_path_sens_kp, f_proc_name_hash, emb_f_proc_name_hash, f_parent_proc_hash, emb_f_parent_proc_hash, f_proc_cwd_hash, emb_f_proc_cwd_hash, f_lineage_bag_hash, emb_f_lineage_bag_hash, f_cmdline_entropy, emb_f_cmdline_entropy, f_cmdline_compress, emb_f_cmdline_compress, f_time_since_parent_exec, emb_f_time_since_parent_exec, f_kp_fd_install_path_sensitivity, emb_f_kp_fd_install_path_sensitivity, f_kp_mmap_path_sensitivity, emb_f_kp_mmap_path_sensitivity, f_kp_tcp_connect_dst_port_bucket, emb_f_kp_tcp_connect_dst_port_bucket, f_kp_tcp_connect_sock_family, emb_f_kp_tcp_connect_sock_family, f_action_family, emb_f_action_family, f_lineage_depth, emb_f_lineage_depth, f_parent_child_pair_hash, emb_f_parent_child_pair_hash, f_root_ancestor_basename_hash, emb_f_root_ancestor_basename_hash, f_process_tree_id_hash, emb_f_process_tree_id_hash, f_delta_t_log_bucket, emb_f_delta_t_log_bucket, f_process_age_log_bucket, emb_f_process_age_log_bucket, f_path_category, emb_f_path_category, f_dst_ip_category, emb_f_dst_ip_category, f_dst_port_category, emb_f_dst_port_category, f_object_category, emb_f_object_category, f_is_procfs_walk, f_uid_eq_parent, f_is_setuid_exec, f_kp_commit_creds_uid_change, f_kp_commit_creds_cap_change, f_kp_udp_sendmsg_dport_eq_53, f_kp_fd_install_fd_int32, f_kp_mmap_prot_uint, f_kp_mprotect_prot_uint, proj_W, proj_b):
    inp = dict(locals())

    idx_cols = [inp[n].reshape(_N).astype(jnp.int32) for (n, _c, _d) in _SMALL]
    idx_mat = jnp.stack(idx_cols, axis=0)
    idx_mat = jnp.pad(idx_mat, ((0, _IDX_PAD - len(_SMALL)), (0, 0)))

    fl_cols = [inp[n].reshape(_N).astype(jnp.float32) for (n, _s) in _FLOATS]
    fl_mat = jnp.stack(fl_cols, axis=0)
    fl_mat = jnp.pad(fl_mat, ((0, 16 - len(_FLOATS)), (0, 0)))

    big_idx = [inp[n].astype(jnp.int32) for (n, _c, _d, _w, _o) in _BIG]
    big_tab = [inp["emb_" + n] for (n, _c, _d, _w, _o) in _BIG]

    bd = jnp.zeros((_OH_PAD, _SDIM_TOT), jnp.float32)
    for (n, c, d), oho, sdo in zip(_SMALL, _OH_OFF, _SDIM_OFF):
        bd = bd.at[oho:oho + c, sdo:sdo + d].set(inp["emb_" + n])
    ws = jnp.concatenate(
        [proj_W[_W_OFF[n]:_W_OFF[n] + d] for (n, _c, d) in _SMALL], axis=0)
    wb = jnp.concatenate(
        [proj_W[_W_OFF[n]:_W_OFF[n] + d] for (n, _c, d, _w, _o) in _BIG], axis=0)
    wf = jnp.concatenate(
        [proj_W[_W_FLOAT_OFF:_W_FLOAT_OFF + len(_FLOATS)],
         jnp.zeros((16 - len(_FLOATS), _D), jnp.float32)], axis=0)
    b2 = proj_b.reshape(1, _D)

    s_np = np.zeros((_IDX_PAD, _OH_PAD), np.float32)
    tgt_np = np.full((1, _OH_PAD), -1.0, np.float32)
    for f, ((n, c, d), oho) in enumerate(zip(_SMALL, _OH_OFF)):
        s_np[f, oho:oho + c] = 1.0
        tgt_np[0, oho:oho + c] = np.arange(c, dtype=np.float32)
    inv_np = np.ones((16, 1), np.float32)
    for f, (n, sc) in enumerate(_FLOATS):
        inv_np[f, 0] = 1.0 / sc
    s_c = jnp.asarray(s_np)
    tgt_c = jnp.asarray(tgt_np)
    inv_c = jnp.asarray(inv_np)

    t_small = pl.pallas_call(
        _fuse_body,
        out_shape=jax.ShapeDtypeStruct((_OH_PAD, _D), jnp.float32),
    )(bd, ws)

    raw = _make_sc_gather()(*big_tab, *big_idx)

    out = pl.pallas_call(
        _tc_body,
        grid=(_N // _TILE,),
        in_specs=[
            pl.BlockSpec((_IDX_PAD, _TILE), lambda i: (0, i)),
            pl.BlockSpec((_TILE, _RAW_COLS), lambda i: (i, 0)),
            pl.BlockSpec((16, _TILE), lambda i: (0, i)),
            pl.BlockSpec((_IDX_PAD, _OH_PAD), lambda i: (0, 0)),
            pl.BlockSpec((1, _OH_PAD), lambda i: (0, 0)),
            pl.BlockSpec((16, 1), lambda i: (0, 0)),
            pl.BlockSpec((_OH_PAD, _D), lambda i: (0, 0)),
            pl.BlockSpec((_RAW_COLS, _D), lambda i: (0, 0)),
            pl.BlockSpec((16, _D), lambda i: (0, 0)),
            pl.BlockSpec((1, _D), lambda i: (0, 0)),
        ],
        out_specs=pl.BlockSpec((_TILE, _D), lambda i: (i, 0)),
        out_shape=jax.ShapeDtypeStruct((_N, _D), jnp.float32),
    )(idx_mat, raw, fl_mat, s_c, tgt_c, inv_c, t_small, wb, wf, b2)

    return out.reshape(_B, _L, _D)


# speedup vs baseline: 2.5661x; 1.0485x over previous
import functools

import numpy as np
import jax
import jax.numpy as jnp
from jax import lax
from jax.experimental import pallas as pl
from jax.experimental.pallas import tpu as pltpu
from jax.experimental.pallas import tpu_sc as plsc

_B, _L = 1024, 200
_N = _B * _L
_D = 128

_EMBED = [
    ("f_event_type", 32, 8), ("f_kprobe_function", 64, 16), ("f_kprobe_policy", 8, 8),
    ("f_kprobe_action", 8, 4), ("f_proc_uid_bucket", 8, 4), ("f_dst_port_bucket", 7, 4),
    ("f_args_length_bucket", 6, 4), ("f_cap_count_bucket", 5, 4), ("f_path_sens_cwd", 4, 8),
    ("f_path_sens_binary", 4, 8), ("f_path_sens_kp", 4, 8), ("f_proc_name_hash", 65536, 32),
    ("f_parent_proc_hash", 65536, 32), ("f_proc_cwd_hash", 16384, 16), ("f_lineage_bag_hash", 16384, 16),
    ("f_cmdline_entropy", 4, 4), ("f_cmdline_compress", 4, 4), ("f_time_since_parent_exec", 8, 4),
    ("f_kp_fd_install_path_sensitivity", 4, 4), ("f_kp_mmap_path_sensitivity", 4, 4),
    ("f_kp_tcp_connect_dst_port_bucket", 7, 4), ("f_kp_tcp_connect_sock_family", 8, 4),
    ("f_action_family", 16, 8), ("f_lineage_depth", 16, 4), ("f_parent_child_pair_hash", 1024, 16),
    ("f_root_ancestor_basename_hash", 1024, 8), ("f_process_tree_id_hash", 4096, 8),
    ("f_delta_t_log_bucket", 10, 4), ("f_process_age_log_bucket", 10, 4), ("f_path_category", 16, 4),
    ("f_dst_ip_category", 8, 4), ("f_dst_port_category", 8, 4), ("f_object_category", 8, 4),
]
_FLOATS = [
    ("f_is_procfs_walk", 1.0), ("f_uid_eq_parent", 1.0), ("f_is_setuid_exec", 1.0),
    ("f_kp_commit_creds_uid_change", 1.0), ("f_kp_commit_creds_cap_change", 1.0),
    ("f_kp_udp_sendmsg_dport_eq_53", 1.0), ("f_kp_fd_install_fd_int32", 1024.0),
    ("f_kp_mmap_prot_uint", 256.0), ("f_kp_mprotect_prot_uint", 256.0),
]
_BIG_NAMES = (
    "f_proc_name_hash", "f_parent_proc_hash", "f_proc_cwd_hash", "f_lineage_bag_hash",
    "f_parent_child_pair_hash", "f_root_ancestor_basename_hash", "f_process_tree_id_hash",
)

_W_OFF = {}
_off = 0
for _n, _c, _d in _EMBED:
    _W_OFF[_n] = _off
    _off += _d
_W_FLOAT_OFF = _off

_SMALL = [(n, c, d) for (n, c, d) in _EMBED if n not in _BIG_NAMES]
_OH_OFF, _SDIM_OFF = [], []
_o1 = _o2 = 0
for _n, _c, _d in _SMALL:
    _OH_OFF.append(_o1)
    _SDIM_OFF.append(_o2)
    _o1 += _c
    _o2 += _d
_OH_TOT = _o1
_SDIM_TOT = _o2
_OH_PAD = 288
_IDX_PAD = 32

_BIG = []
_o3 = 0
for _n in _BIG_NAMES:
    _c, _d = next((c, d) for (nm, c, d) in _EMBED if nm == _n)
    _BIG.append((_n, _c, _d, _d, _o3))
    _o3 += _d
_RAW_COLS = _o3

_TILE = 4096
_LANES = 128
_GROUPS = _N // _LANES


def _fuse_body(bd_ref, ws_ref, t_ref):
    t_ref[...] = jnp.dot(bd_ref[...], ws_ref[...], preferred_element_type=jnp.float32)


_DN0 = (((0,), (0,)), ((), ()))


def _tc_body(idx_ref, raw_ref, fl_ref, s_ref, tgt_ref,
             inv_ref, t_ref, wb_ref, wf_ref, b_ref, out_ref):
    idxf = idx_ref[...].astype(jnp.float32)
    g = lax.dot_general(idxf, s_ref[...], _DN0,
                        preferred_element_type=jnp.float32)
    oh = jnp.where(jnp.abs(g - tgt_ref[...]) < 0.5, 1.0, 0.0)
    fl = fl_ref[...] * inv_ref[...]
    acc = jnp.dot(oh, t_ref[...], preferred_element_type=jnp.float32)
    acc = acc + jnp.dot(raw_ref[...], wb_ref[...],
                        preferred_element_type=jnp.float32)
    acc = acc + lax.dot_general(fl, wf_ref[...], _DN0,
                                preferred_element_type=jnp.float32)
    out_ref[...] = acc + b_ref[...]


@functools.lru_cache(maxsize=1)
def _make_sc_gather():
    info = plsc.get_sparse_core_info()
    nc, ns = info.num_cores, info.num_subcores
    nw = nc * ns
    tpw = _N // nw
    rpw = _B // nw
    ch_r = 4
    half = ch_r * _L
    n_bodies = rpw // (2 * ch_r)
    runs = ((0, 104), (104, 96))
    mesh = plsc.VectorSubcoreMesh(core_axis_name="c", subcore_axis_name="s")

    @functools.partial(
        pl.kernel, mesh=mesh,
        compiler_params=pltpu.CompilerParams(use_tc_tiling_on_sc=False),
        out_type=jax.ShapeDtypeStruct((_N, _RAW_COLS), jnp.float32),
        scratch_types=[
            pltpu.VMEM((rpw, _L), jnp.int32),
            pltpu.VMEM((half, 32), jnp.float32),
            pltpu.VMEM((half, 32), jnp.float32),
            pltpu.VMEM((half, 16), jnp.float32),
            pltpu.VMEM((half, 16), jnp.float32),
            pltpu.VMEM((half, 8), jnp.float32),
            pltpu.VMEM((half, 8), jnp.float32),
            pltpu.SemaphoreType.DMA,
            pltpu.SemaphoreType.DMA,
        ],
    )
    def sc_gather(t0, t1, t2, t3, t4, t5, t6, i0, i1, i2, i3, i4, i5, i6,
                  out_ref, idx_v, ra32, rb32, ra16, rb16, ra8, rb8,
                  sem_g, sem_o):
        wid = lax.axis_index("s") * nc + lax.axis_index("c")
        tabs = (t0, t1, t2, t3, t4, t5, t6)
        idxs = (i0, i1, i2, i3, i4, i5, i6)
        row0 = wid * rpw
        tok0 = row0 * _L
        for f, (_nm, _card, _dim, w, col) in enumerate(_BIG):
            tab, idxh = tabs[f], idxs[f]
            ra, rb = {32: (ra32, rb32), 16: (ra16, rb16), 8: (ra8, rb8)}[w]
            pltpu.sync_copy(idxh.at[pl.ds(row0, rpw)], idx_v)

            def fire_half(r_base, buf, tab=tab):
                cps = []
                for r2 in range(ch_r):
                    for (c0, cl) in runs:
                        cps.append(pltpu.async_copy(
                            tab.at[idx_v.at[r_base + r2, pl.ds(c0, cl)]],
                            buf.at[pl.ds(r2 * _L + c0, cl)],
                            sem_g,
                        ))
                return cps

            def body(k, carry, tab=tab, ra=ra, rb=rb, col=col, w=w,
                     fire_half=fire_half):
                ra_row = k * 2 * ch_r
                rb_row = ra_row + ch_r
                cps_a = fire_half(ra_row, ra)
                cps_b = fire_half(rb_row, rb)
                for cp in cps_a:
                    cp.wait()
                st_a = pltpu.async_copy(
                    ra,
                    out_ref.at[pl.ds(tok0 + ra_row * _L, half), pl.ds(col, w)],
                    sem_o)
                for cp in cps_b:
                    cp.wait()
                st_b = pltpu.async_copy(
                    rb,
                    out_ref.at[pl.ds(tok0 + rb_row * _L, half), pl.ds(col, w)],
                    sem_o)
                st_a.wait()
                st_b.wait()
                return carry

            lax.fori_loop(0, n_bodies, body, 0)

    return sc_gather


def kernel(f_event_type, emb_f_event_type, f_kprobe_function, emb_f_kprobe_function, f_kprobe_policy, emb_f_kprobe_policy, f_kprobe_action, emb_f_kprobe_action, f_proc_uid_bucket, emb_f_proc_uid_bucket, f_dst_port_bucket, emb_f_dst_port_bucket, f_args_length_bucket, emb_f_args_length_bucket, f_cap_count_bucket, emb_f_cap_count_bucket, f_path_sens_cwd, emb_f_path_sens_cwd, f_path_sens_binary, emb_f_path_sens_binary, f_path_sens_kp, emb_f_path_sens_kp, f_proc_name_hash, emb_f_proc_name_hash, f_parent_proc_hash, emb_f_parent_proc_hash, f_proc_cwd_hash, emb_f_proc_cwd_hash, f_lineage_bag_hash, emb_f_lineage_bag_hash, f_cmdline_entropy, emb_f_cmdline_entropy, f_cmdline_compress, emb_f_cmdline_compress, f_time_since_parent_exec, emb_f_time_since_parent_exec, f_kp_fd_install_path_sensitivity, emb_f_kp_fd_install_path_sensitivity, f_kp_mmap_path_sensitivity, emb_f_kp_mmap_path_sensitivity, f_kp_tcp_connect_dst_port_bucket, emb_f_kp_tcp_connect_dst_port_bucket, f_kp_tcp_connect_sock_family, emb_f_kp_tcp_connect_sock_family, f_action_family, emb_f_action_family, f_lineage_depth, emb_f_lineage_depth, f_parent_child_pair_hash, emb_f_parent_child_pair_hash, f_root_ancestor_basename_hash, emb_f_root_ancestor_basename_hash, f_process_tree_id_hash, emb_f_process_tree_id_hash, f_delta_t_log_bucket, emb_f_delta_t_log_bucket, f_process_age_log_bucket, emb_f_process_age_log_bucket, f_path_category, emb_f_path_category, f_dst_ip_category, emb_f_dst_ip_category, f_dst_port_category, emb_f_dst_port_category, f_object_category, emb_f_object_category, f_is_procfs_walk, f_uid_eq_parent, f_is_setuid_exec, f_kp_commit_creds_uid_change, f_kp_commit_creds_cap_change, f_kp_udp_sendmsg_dport_eq_53, f_kp_fd_install_fd_int32, f_kp_mmap_prot_uint, f_kp_mprotect_prot_uint, proj_W, proj_b):
    inp = dict(locals())

    idx_cols = [inp[n].reshape(_N).astype(jnp.int32) for (n, _c, _d) in _SMALL]
    idx_mat = jnp.stack(idx_cols, axis=0)
    idx_mat = jnp.pad(idx_mat, ((0, _IDX_PAD - len(_SMALL)), (0, 0)))

    fl_cols = [inp[n].reshape(_N).astype(jnp.float32) for (n, _s) in _FLOATS]
    fl_mat = jnp.stack(fl_cols, axis=0)
    fl_mat = jnp.pad(fl_mat, ((0, 16 - len(_FLOATS)), (0, 0)))

    big_idx = [inp[n].astype(jnp.int32) for (n, _c, _d, _w, _o) in _BIG]
    big_tab = [inp["emb_" + n] for (n, _c, _d, _w, _o) in _BIG]

    bd = jnp.zeros((_OH_PAD, _SDIM_TOT), jnp.float32)
    for (n, c, d), oho, sdo in zip(_SMALL, _OH_OFF, _SDIM_OFF):
        bd = bd.at[oho:oho + c, sdo:sdo + d].set(inp["emb_" + n])
    ws = jnp.concatenate(
        [proj_W[_W_OFF[n]:_W_OFF[n] + d] for (n, _c, d) in _SMALL], axis=0)
    wb = jnp.concatenate(
        [proj_W[_W_OFF[n]:_W_OFF[n] + d] for (n, _c, d, _w, _o) in _BIG], axis=0)
    wf = jnp.concatenate(
        [proj_W[_W_FLOAT_OFF:_W_FLOAT_OFF + len(_FLOATS)],
         jnp.zeros((16 - len(_FLOATS), _D), jnp.float32)], axis=0)
    b2 = proj_b.reshape(1, _D)

    s_np = np.zeros((_IDX_PAD, _OH_PAD), np.float32)
    tgt_np = np.full((1, _OH_PAD), -1.0, np.float32)
    for f, ((n, c, d), oho) in enumerate(zip(_SMALL, _OH_OFF)):
        s_np[f, oho:oho + c] = 1.0
        tgt_np[0, oho:oho + c] = np.arange(c, dtype=np.float32)
    inv_np = np.ones((16, 1), np.float32)
    for f, (n, sc) in enumerate(_FLOATS):
        inv_np[f, 0] = 1.0 / sc
    s_c = jnp.asarray(s_np)
    tgt_c = jnp.asarray(tgt_np)
    inv_c = jnp.asarray(inv_np)

    t_small = pl.pallas_call(
        _fuse_body,
        out_shape=jax.ShapeDtypeStruct((_OH_PAD, _D), jnp.float32),
    )(bd, ws)

    raw = _make_sc_gather()(*big_tab, *big_idx)

    out = pl.pallas_call(
        _tc_body,
        grid=(_N // _TILE,),
        in_specs=[
            pl.BlockSpec((_IDX_PAD, _TILE), lambda i: (0, i)),
            pl.BlockSpec((_TILE, _RAW_COLS), lambda i: (i, 0)),
            pl.BlockSpec((16, _TILE), lambda i: (0, i)),
            pl.BlockSpec((_IDX_PAD, _OH_PAD), lambda i: (0, 0)),
            pl.BlockSpec((1, _OH_PAD), lambda i: (0, 0)),
            pl.BlockSpec((16, 1), lambda i: (0, 0)),
            pl.BlockSpec((_OH_PAD, _D), lambda i: (0, 0)),
            pl.BlockSpec((_RAW_COLS, _D), lambda i: (0, 0)),
            pl.BlockSpec((16, _D), lambda i: (0, 0)),
            pl.BlockSpec((1, _D), lambda i: (0, 0)),
        ],
        out_specs=pl.BlockSpec((_TILE, _D), lambda i: (i, 0)),
        out_shape=jax.ShapeDtypeStruct((_N, _D), jnp.float32),
    )(idx_mat, raw, fl_mat, s_c, tgt_c, inv_c, t_small, wb, wf, b2)

    return out.reshape(_B, _L, _D)


# speedup vs baseline: 2.6085x; 1.0165x over previous
import functools

import numpy as np
import jax
import jax.numpy as jnp
from jax import lax
from jax.experimental import pallas as pl
from jax.experimental.pallas import tpu as pltpu
from jax.experimental.pallas import tpu_sc as plsc

_B, _L = 1024, 200
_N = _B * _L
_D = 128

_EMBED = [
    ("f_event_type", 32, 8), ("f_kprobe_function", 64, 16), ("f_kprobe_policy", 8, 8),
    ("f_kprobe_action", 8, 4), ("f_proc_uid_bucket", 8, 4), ("f_dst_port_bucket", 7, 4),
    ("f_args_length_bucket", 6, 4), ("f_cap_count_bucket", 5, 4), ("f_path_sens_cwd", 4, 8),
    ("f_path_sens_binary", 4, 8), ("f_path_sens_kp", 4, 8), ("f_proc_name_hash", 65536, 32),
    ("f_parent_proc_hash", 65536, 32), ("f_proc_cwd_hash", 16384, 16), ("f_lineage_bag_hash", 16384, 16),
    ("f_cmdline_entropy", 4, 4), ("f_cmdline_compress", 4, 4), ("f_time_since_parent_exec", 8, 4),
    ("f_kp_fd_install_path_sensitivity", 4, 4), ("f_kp_mmap_path_sensitivity", 4, 4),
    ("f_kp_tcp_connect_dst_port_bucket", 7, 4), ("f_kp_tcp_connect_sock_family", 8, 4),
    ("f_action_family", 16, 8), ("f_lineage_depth", 16, 4), ("f_parent_child_pair_hash", 1024, 16),
    ("f_root_ancestor_basename_hash", 1024, 8), ("f_process_tree_id_hash", 4096, 8),
    ("f_delta_t_log_bucket", 10, 4), ("f_process_age_log_bucket", 10, 4), ("f_path_category", 16, 4),
    ("f_dst_ip_category", 8, 4), ("f_dst_port_category", 8, 4), ("f_object_category", 8, 4),
]
_FLOATS = [
    ("f_is_procfs_walk", 1.0), ("f_uid_eq_parent", 1.0), ("f_is_setuid_exec", 1.0),
    ("f_kp_commit_creds_uid_change", 1.0), ("f_kp_commit_creds_cap_change", 1.0),
    ("f_kp_udp_sendmsg_dport_eq_53", 1.0), ("f_kp_fd_install_fd_int32", 1024.0),
    ("f_kp_mmap_prot_uint", 256.0), ("f_kp_mprotect_prot_uint", 256.0),
]
_BIG_NAMES = (
    "f_proc_name_hash", "f_parent_proc_hash", "f_proc_cwd_hash", "f_lineage_bag_hash",
    "f_parent_child_pair_hash", "f_root_ancestor_basename_hash", "f_process_tree_id_hash",
)

_W_OFF = {}
_off = 0
for _n, _c, _d in _EMBED:
    _W_OFF[_n] = _off
    _off += _d
_W_FLOAT_OFF = _off

_SMALL = [(n, c, d) for (n, c, d) in _EMBED if n not in _BIG_NAMES]
_OH_OFF, _SDIM_OFF = [], []
_o1 = _o2 = 0
for _n, _c, _d in _SMALL:
    _OH_OFF.append(_o1)
    _SDIM_OFF.append(_o2)
    _o1 += _c
    _o2 += _d
_OH_TOT = _o1
_SDIM_TOT = _o2
_OH_PAD = 288
_IDX_PAD = 32

_BIG = []
_o3 = 0
for _n in _BIG_NAMES:
    _c, _d = next((c, d) for (nm, c, d) in _EMBED if nm == _n)
    _BIG.append((_n, _c, _d, _d, _o3))
    _o3 += _d
_RAW_COLS = _o3

_TILE = 8192
_LANES = 128
_GROUPS = _N // _LANES


def _fuse_body(bd_ref, ws_ref, t_ref):
    t_ref[...] = jnp.dot(bd_ref[...], ws_ref[...], preferred_element_type=jnp.float32)


_DN0 = (((0,), (0,)), ((), ()))


def _tc_body(idx_ref, raw_ref, fl_ref, s_ref, tgt_ref,
             inv_ref, t_ref, wb_ref, wf_ref, b_ref, out_ref):
    idxf = idx_ref[...].astype(jnp.float32)
    g = lax.dot_general(idxf, s_ref[...], _DN0,
                        preferred_element_type=jnp.float32)
    oh = jnp.where(jnp.abs(g - tgt_ref[...]) < 0.5, 1.0, 0.0)
    fl = fl_ref[...] * inv_ref[...]
    acc = jnp.dot(oh, t_ref[...], preferred_element_type=jnp.float32)
    acc = acc + jnp.dot(raw_ref[...], wb_ref[...],
                        preferred_element_type=jnp.float32)
    acc = acc + lax.dot_general(fl, wf_ref[...], _DN0,
                                preferred_element_type=jnp.float32)
    out_ref[...] = acc + b_ref[...]


@functools.lru_cache(maxsize=1)
def _make_sc_gather():
    info = plsc.get_sparse_core_info()
    nc, ns = info.num_cores, info.num_subcores
    nw = nc * ns
    tpw = _N // nw
    rpw = _B // nw
    ch_r = 4
    half = ch_r * _L
    n_bodies = rpw // (2 * ch_r)
    runs = ((0, 104), (104, 96))
    mesh = plsc.VectorSubcoreMesh(core_axis_name="c", subcore_axis_name="s")

    @functools.partial(
        pl.kernel, mesh=mesh,
        compiler_params=pltpu.CompilerParams(use_tc_tiling_on_sc=False),
        out_type=jax.ShapeDtypeStruct((_N, _RAW_COLS), jnp.float32),
        scratch_types=[
            pltpu.VMEM((rpw, _L), jnp.int32),
            pltpu.VMEM((half, 32), jnp.float32),
            pltpu.VMEM((half, 32), jnp.float32),
            pltpu.VMEM((half, 16), jnp.float32),
            pltpu.VMEM((half, 16), jnp.float32),
            pltpu.VMEM((half, 8), jnp.float32),
            pltpu.VMEM((half, 8), jnp.float32),
            pltpu.SemaphoreType.DMA,
            pltpu.SemaphoreType.DMA,
        ],
    )
    def sc_gather(t0, t1, t2, t3, t4, t5, t6, i0, i1, i2, i3, i4, i5, i6,
                  out_ref, idx_v, ra32, rb32, ra16, rb16, ra8, rb8,
                  sem_g, sem_o):
        wid = lax.axis_index("s") * nc + lax.axis_index("c")
        tabs = (t0, t1, t2, t3, t4, t5, t6)
        idxs = (i0, i1, i2, i3, i4, i5, i6)
        row0 = wid * rpw
        tok0 = row0 * _L
        for f, (_nm, _card, _dim, w, col) in enumerate(_BIG):
            tab, idxh = tabs[f], idxs[f]
            ra, rb = {32: (ra32, rb32), 16: (ra16, rb16), 8: (ra8, rb8)}[w]
            pltpu.sync_copy(idxh.at[pl.ds(row0, rpw)], idx_v)

            def fire_half(r_base, buf, tab=tab):
                cps = []
                for r2 in range(ch_r):
                    for (c0, cl) in runs:
                        cps.append(pltpu.async_copy(
                            tab.at[idx_v.at[r_base + r2, pl.ds(c0, cl)]],
                            buf.at[pl.ds(r2 * _L + c0, cl)],
                            sem_g,
                        ))
                return cps

            def body(k, carry, tab=tab, ra=ra, rb=rb, col=col, w=w,
                     fire_half=fire_half):
                ra_row = k * 2 * ch_r
                rb_row = ra_row + ch_r
                cps_a = fire_half(ra_row, ra)
                cps_b = fire_half(rb_row, rb)
                for cp in cps_a:
                    cp.wait()
                st_a = pltpu.async_copy(
                    ra,
                    out_ref.at[pl.ds(tok0 + ra_row * _L, half), pl.ds(col, w)],
                    sem_o)
                for cp in cps_b:
                    cp.wait()
                st_b = pltpu.async_copy(
                    rb,
                    out_ref.at[pl.ds(tok0 + rb_row * _L, half), pl.ds(col, w)],
                    sem_o)
                st_a.wait()
                st_b.wait()
                return carry

            lax.fori_loop(0, n_bodies, body, 0)

    return sc_gather


def kernel(f_event_type, emb_f_event_type, f_kprobe_function, emb_f_kprobe_function, f_kprobe_policy, emb_f_kprobe_policy, f_kprobe_action, emb_f_kprobe_action, f_proc_uid_bucket, emb_f_proc_uid_bucket, f_dst_port_bucket, emb_f_dst_port_bucket, f_args_length_bucket, emb_f_args_length_bucket, f_cap_count_bucket, emb_f_cap_count_bucket, f_path_sens_cwd, emb_f_path_sens_cwd, f_path_sens_binary, emb_f_path_sens_binary, f_path_sens_kp, emb_f_path_sens_kp, f_proc_name_hash, emb_f_proc_name_hash, f_parent_proc_hash, emb_f_parent_proc_hash, f_proc_cwd_hash, emb_f_proc_cwd_hash, f_lineage_bag_hash, emb_f_lineage_bag_hash, f_cmdline_entropy, emb_f_cmdline_entropy, f_cmdline_compress, emb_f_cmdline_compress, f_time_since_parent_exec, emb_f_time_since_parent_exec, f_kp_fd_install_path_sensitivity, emb_f_kp_fd_install_path_sensitivity, f_kp_mmap_path_sensitivity, emb_f_kp_mmap_path_sensitivity, f_kp_tcp_connect_dst_port_bucket, emb_f_kp_tcp_connect_dst_port_bucket, f_kp_tcp_connect_sock_family, emb_f_kp_tcp_connect_sock_family, f_action_family, emb_f_action_family, f_lineage_depth, emb_f_lineage_depth, f_parent_child_pair_hash, emb_f_parent_child_pair_hash, f_root_ancestor_basename_hash, emb_f_root_ancestor_basename_hash, f_process_tree_id_hash, emb_f_process_tree_id_hash, f_delta_t_log_bucket, emb_f_delta_t_log_bucket, f_process_age_log_bucket, emb_f_process_age_log_bucket, f_path_category, emb_f_path_category, f_dst_ip_category, emb_f_dst_ip_category, f_dst_port_category, emb_f_dst_port_category, f_object_category, emb_f_object_category, f_is_procfs_walk, f_uid_eq_parent, f_is_setuid_exec, f_kp_commit_creds_uid_change, f_kp_commit_creds_cap_change, f_kp_udp_sendmsg_dport_eq_53, f_kp_fd_install_fd_int32, f_kp_mmap_prot_uint, f_kp_mprotect_prot_uint, proj_W, proj_b):
    inp = dict(locals())

    idx_cols = [inp[n].reshape(_N).astype(jnp.int32) for (n, _c, _d) in _SMALL]
    idx_mat = jnp.stack(idx_cols, axis=0)
    idx_mat = jnp.pad(idx_mat, ((0, _IDX_PAD - len(_SMALL)), (0, 0)))

    fl_cols = [inp[n].reshape(_N).astype(jnp.float32) for (n, _s) in _FLOATS]
    fl_mat = jnp.stack(fl_cols, axis=0)
    fl_mat = jnp.pad(fl_mat, ((0, 16 - len(_FLOATS)), (0, 0)))

    big_idx = [inp[n].astype(jnp.int32) for (n, _c, _d, _w, _o) in _BIG]
    big_tab = [inp["emb_" + n] for (n, _c, _d, _w, _o) in _BIG]

    bd = jnp.zeros((_OH_PAD, _SDIM_TOT), jnp.float32)
    for (n, c, d), oho, sdo in zip(_SMALL, _OH_OFF, _SDIM_OFF):
        bd = bd.at[oho:oho + c, sdo:sdo + d].set(inp["emb_" + n])
    ws = jnp.concatenate(
        [proj_W[_W_OFF[n]:_W_OFF[n] + d] for (n, _c, d) in _SMALL], axis=0)
    wb = jnp.concatenate(
        [proj_W[_W_OFF[n]:_W_OFF[n] + d] for (n, _c, d, _w, _o) in _BIG], axis=0)
    wf = jnp.concatenate(
        [proj_W[_W_FLOAT_OFF:_W_FLOAT_OFF + len(_FLOATS)],
         jnp.zeros((16 - len(_FLOATS), _D), jnp.float32)], axis=0)
    b2 = proj_b.reshape(1, _D)

    s_np = np.zeros((_IDX_PAD, _OH_PAD), np.float32)
    tgt_np = np.full((1, _OH_PAD), -1.0, np.float32)
    for f, ((n, c, d), oho) in enumerate(zip(_SMALL, _OH_OFF)):
        s_np[f, oho:oho + c] = 1.0
        tgt_np[0, oho:oho + c] = np.arange(c, dtype=np.float32)
    inv_np = np.ones((16, 1), np.float32)
    for f, (n, sc) in enumerate(_FLOATS):
        inv_np[f, 0] = 1.0 / sc
    s_c = jnp.asarray(s_np)
    tgt_c = jnp.asarray(tgt_np)
    inv_c = jnp.asarray(inv_np)

    t_small = pl.pallas_call(
        _fuse_body,
        out_shape=jax.ShapeDtypeStruct((_OH_PAD, _D), jnp.float32),
    )(bd, ws)

    raw = _make_sc_gather()(*big_tab, *big_idx)

    out = pl.pallas_call(
        _tc_body,
        grid=(_N // _TILE,),
        in_specs=[
            pl.BlockSpec((_IDX_PAD, _TILE), lambda i: (0, i)),
            pl.BlockSpec((_TILE, _RAW_COLS), lambda i: (i, 0)),
            pl.BlockSpec((16, _TILE), lambda i: (0, i)),
            pl.BlockSpec((_IDX_PAD, _OH_PAD), lambda i: (0, 0)),
            pl.BlockSpec((1, _OH_PAD), lambda i: (0, 0)),
            pl.BlockSpec((16, 1), lambda i: (0, 0)),
            pl.BlockSpec((_OH_PAD, _D), lambda i: (0, 0)),
            pl.BlockSpec((_RAW_COLS, _D), lambda i: (0, 0)),
            pl.BlockSpec((16, _D), lambda i: (0, 0)),
            pl.BlockSpec((1, _D), lambda i: (0, 0)),
        ],
        out_specs=pl.BlockSpec((_TILE, _D), lambda i: (i, 0)),
        out_shape=jax.ShapeDtypeStruct((_N, _D), jnp.float32),
    )(idx_mat, raw, fl_mat, s_c, tgt_c, inv_c, t_small, wb, wf, b2)

    return out.reshape(_B, _L, _D)
